# Initial kernel scaffold; baseline (speedup 1.0000x reference)
import jax
import jax.numpy as jnp
from jax.experimental import pallas as pl


def kernel(x_adj, edge_index, neg_edge_index, Wt, bt, W10, W11, b1, Wm0, Wm1, bm, Wl0, Wl1, bl):
    raise NotImplementedError("write your pallas kernel here")


# speedup vs baseline: 4.4859x; 4.4859x over previous
import functools
import jax
import jax.numpy as jnp
from jax import lax
from jax.experimental import pallas as pl
from jax.experimental.pallas import tpu as pltpu
from jax.experimental.pallas import tpu_sc as plsc

N = 10000
E = 320000
HID = 128
ZD = 32
EPS = 1e-15
MAX_LOGSTD = 10.0

NC, NS, L = 2, 16, 16
NW = NC * NS
EPW = E // NW
CH = 80
NCH = EPW // CH
PADROW = N
NPAD_A = 10112
RB = 632
RB_LAST = N - (NS - 1) * RB


@functools.cache
def _sc_mesh():
    return plsc.VectorSubcoreMesh(core_axis_name="c", subcore_axis_name="s",
                                  num_cores=NC, num_subcores=NS)


def _fill_zeros8(zb):
    for i in range(8):
        for c in range(8):
            zb[i, pl.ds(c * L, L)] = jnp.zeros((L,), jnp.float32)


def _zero_rows(zb, acc, start, nrows):
    def cp(i, _):
        off = pl.multiple_of(start + i * 8, 8)
        pltpu.sync_copy(zb, acc.at[pl.ds(off, 8)])
        return _
    lax.fori_loop(0, nrows // 8, cp, None)


def _edge_prep_body(row_hbm, col_hbm, deg_hbm, remap_hbm,
                    rowv, colv, rmv, ones, zb, dacc, sem):
    cid = lax.axis_index("c")
    sid = lax.axis_index("s")
    wid = sid * NC + cid
    RBA = NPAD_A // NS

    _fill_zeros8(zb)
    _zero_rows(zb, dacc, sid * RBA, RBA)

    def of(i, _):
        for c in range(8):
            ones[i, pl.ds(c * L, L)] = jnp.full((L,), 1.0, jnp.float32)
        return _
    lax.fori_loop(0, CH, of, None)
    plsc.subcore_barrier()

    def chunk(j, _):
        base = wid * EPW + j * CH
        pltpu.sync_copy(row_hbm.at[pl.ds(base, CH)], rowv)
        pltpu.sync_copy(col_hbm.at[pl.ds(base, CH)], colv)

        def vec(k, _):
            r = rowv[pl.ds(k * L, L)]
            c = colv[pl.ds(k * L, L)]
            rmv[pl.ds(k * L, L)] = jnp.where(r == c, PADROW, r).astype(jnp.int32)
            return _
        lax.fori_loop(0, CH // L, vec, None)
        pltpu.sync_copy(ones, dacc.at[rmv], add=True)
        pltpu.sync_copy(rmv, remap_hbm.at[pl.ds(base, CH)])
        return _
    lax.fori_loop(0, NCH, chunk, None)
    plsc.subcore_barrier()
    off = pl.multiple_of(sid * RBA, 8)
    pltpu.sync_copy(dacc.at[pl.ds(off, RBA)], deg_hbm.at[cid, pl.ds(off, RBA)])


def _edge_prep(row, col):
    return pl.kernel(
        _edge_prep_body,
        out_type=(
            jax.ShapeDtypeStruct((NC, NPAD_A, HID), jnp.float32),
            jax.ShapeDtypeStruct((E,), jnp.int32),
        ),
        mesh=_sc_mesh(),
        scratch_types=[
            pltpu.VMEM((CH,), jnp.int32),
            pltpu.VMEM((CH,), jnp.int32),
            pltpu.VMEM((CH,), jnp.int32),
            pltpu.VMEM((CH, HID), jnp.float32),
            pltpu.VMEM((8, HID), jnp.float32),
            pltpu.VMEM_SHARED((NPAD_A, HID), jnp.float32),
            pltpu.SemaphoreType.DMA,
        ],
    )(row, col)


def _scatter_body(table_hbm, remap_hbm, col_hbm, out_hbm,
                  iv, cv, rows, zb, acc, sem):
    cid = lax.axis_index("c")
    sid = lax.axis_index("s")
    wid = sid * NC + cid

    _fill_zeros8(zb)

    @pl.when(sid < NS - 1)
    def _():
        _zero_rows(zb, acc, sid * RB, RB)

    @pl.when(sid == NS - 1)
    def _():
        _zero_rows(zb, acc, (NS - 1) * RB, RB_LAST)

    plsc.subcore_barrier()

    def chunk(j, _):
        base = wid * EPW + j * CH
        pltpu.sync_copy(remap_hbm.at[pl.ds(base, CH)], iv)
        pltpu.sync_copy(col_hbm.at[pl.ds(base, CH)], cv)
        pltpu.async_copy(table_hbm.at[iv], rows, sem).wait()
        pltpu.sync_copy(rows, acc.at[cv], add=True)
        return _
    lax.fori_loop(0, NCH, chunk, None)
    plsc.subcore_barrier()

    @pl.when(sid < NS - 1)
    def _():
        off = pl.multiple_of(sid * RB, 8)
        pltpu.sync_copy(acc.at[pl.ds(off, RB)], out_hbm.at[cid, pl.ds(off, RB)])

    @pl.when(sid == NS - 1)
    def _():
        off = (NS - 1) * RB
        pltpu.sync_copy(acc.at[pl.ds(off, RB_LAST)],
                        out_hbm.at[cid, pl.ds(off, RB_LAST)])


def _scatter(table_pad, remap, col):
    return pl.kernel(
        _scatter_body,
        out_type=jax.ShapeDtypeStruct((NC, N, HID), jnp.float32),
        mesh=_sc_mesh(),
        scratch_types=[
            pltpu.VMEM((CH,), jnp.int32),
            pltpu.VMEM((CH,), jnp.int32),
            pltpu.VMEM((CH, HID), jnp.float32),
            pltpu.VMEM((8, HID), jnp.float32),
            pltpu.VMEM_SHARED((N, HID), jnp.float32),
            pltpu.SemaphoreType.DMA,
        ],
    )(table_pad, remap, col)


def _recon_body(z_hbm, e0_hbm, e1_hbm, n0_hbm, n1_hbm, qp_hbm, qn_hbm,
                i0, i1, za, zb, qv, sem):
    cid = lax.axis_index("c")
    sid = lax.axis_index("s")
    wid = sid * NC + cid

    def do_side(idx0_hbm, idx1_hbm, q_hbm, j):
        base = wid * EPW + j * CH
        pltpu.sync_copy(idx0_hbm.at[pl.ds(base, CH)], i0)
        pltpu.sync_copy(idx1_hbm.at[pl.ds(base, CH)], i1)
        pltpu.async_copy(z_hbm.at[i0], za, sem).wait()
        pltpu.async_copy(z_hbm.at[i1], zb, sem).wait()

        def dot(e, _):
            a0 = za[e, pl.ds(0, L)]
            a1 = za[e, pl.ds(L, L)]
            b0 = zb[e, pl.ds(0, L)]
            b1 = zb[e, pl.ds(L, L)]
            qv[e, pl.ds(0, L)] = a0 * b0 + a1 * b1
            return _
        lax.fori_loop(0, CH, dot, None)
        pltpu.sync_copy(qv, q_hbm.at[pl.ds(base, CH)])

    def chunk(j, _):
        do_side(e0_hbm, e1_hbm, qp_hbm, j)
        do_side(n0_hbm, n1_hbm, qn_hbm, j)
        return _
    lax.fori_loop(0, NCH, chunk, None)


def _recon(z128, e0, e1, n0, n1):
    return pl.kernel(
        _recon_body,
        out_type=(
            jax.ShapeDtypeStruct((E, L), jnp.float32),
            jax.ShapeDtypeStruct((E, L), jnp.float32),
        ),
        mesh=_sc_mesh(),
        scratch_types=[
            pltpu.VMEM((CH,), jnp.int32),
            pltpu.VMEM((CH,), jnp.int32),
            pltpu.VMEM((CH, HID), jnp.float32),
            pltpu.VMEM((CH, HID), jnp.float32),
            pltpu.VMEM((CH, L), jnp.float32),
            pltpu.SemaphoreType.DMA,
        ],
    )(z128, e0, e1, n0, n1)


def _big_body(x_ref, wt_ref, bt_ref, degt_ref, h_ref, g_ref, dinv_ref):
    acc = jax.lax.dot_general(
        x_ref[...], wt_ref[...], (((1,), (0,)), ((), ())),
        preferred_element_type=jnp.float32)
    deg = jnp.sum(degt_ref[...], axis=1, keepdims=True)
    pos = deg > 0
    dinv = jnp.where(pos, lax.rsqrt(jnp.where(pos, deg, 1.0)), 0.0)
    h = jax.nn.relu(acc + bt_ref[...])
    h_ref[...] = h
    g_ref[...] = h * dinv
    dinv_ref[...] = dinv


def _big(x_adj, Wt, bt, degT):
    BM = 400
    return pl.pallas_call(
        _big_body,
        grid=(N // BM,),
        in_specs=[
            pl.BlockSpec((BM, N), lambda i: (i, 0)),
            pl.BlockSpec((N, HID), lambda i: (0, 0)),
            pl.BlockSpec((1, HID), lambda i: (0, 0)),
            pl.BlockSpec((BM, NC), lambda i: (i, 0)),
        ],
        out_specs=[
            pl.BlockSpec((BM, HID), lambda i: (i, 0)),
            pl.BlockSpec((BM, HID), lambda i: (i, 0)),
            pl.BlockSpec((BM, 1), lambda i: (i, 0)),
        ],
        out_shape=[
            jax.ShapeDtypeStruct((N, HID), jnp.float32),
            jax.ShapeDtypeStruct((N, HID), jnp.float32),
            jax.ShapeDtypeStruct((N, 1), jnp.float32),
        ],
        compiler_params=pltpu.CompilerParams(
            dimension_semantics=("arbitrary",)),
    )(x_adj, Wt, bt, degT)


def _mid_body(h_ref, p0_ref, p1_ref, dinv_ref, w10_ref, w11_ref, b1_ref,
              h1_ref, g1_ref):
    HO = 2 * ZD
    dinv = dinv_ref[...]
    tx1 = -dinv * (p0_ref[...] + p1_ref[...])
    h1 = jnp.dot(h_ref[...], w10_ref[...], preferred_element_type=jnp.float32)
    h1 = h1 + jnp.dot(tx1, w11_ref[...], preferred_element_type=jnp.float32)
    h1 = jax.nn.relu(h1 + b1_ref[...])
    h1_ref[...] = h1
    g1_ref[:, :HO] = h1 * dinv
    g1_ref[:, HO:] = jnp.zeros_like(g1_ref[:, HO:])


def _mid(h, p0, p1, dinv, W10, W11, b1):
    BM = 1000
    HO = 2 * ZD
    return pl.pallas_call(
        _mid_body,
        grid=(N // BM,),
        in_specs=[
            pl.BlockSpec((BM, HID), lambda i: (i, 0)),
            pl.BlockSpec((BM, HID), lambda i: (i, 0)),
            pl.BlockSpec((BM, HID), lambda i: (i, 0)),
            pl.BlockSpec((BM, 1), lambda i: (i, 0)),
            pl.BlockSpec((HID, HO), lambda i: (0, 0)),
            pl.BlockSpec((HID, HO), lambda i: (0, 0)),
            pl.BlockSpec((1, HO), lambda i: (0, 0)),
        ],
        out_specs=[
            pl.BlockSpec((BM, HO), lambda i: (i, 0)),
            pl.BlockSpec((BM, HID), lambda i: (i, 0)),
        ],
        out_shape=[
            jax.ShapeDtypeStruct((N, HO), jnp.float32),
            jax.ShapeDtypeStruct((N, HID), jnp.float32),
        ],
        compiler_params=pltpu.CompilerParams(
            dimension_semantics=("arbitrary",)),
    )(h, p0, p1, dinv, W10, W11, b1)


def _heads_body(h1_ref, q0_ref, q1_ref, dinv_ref, wm0_ref, wm1_ref, bm_ref,
                wl0_ref, wl1_ref, bl_ref, eps_ref, z_ref, z128_ref, kl_ref):
    tx2 = -dinv_ref[...] * (q0_ref[...] + q1_ref[...])
    h1 = h1_ref[...]
    mu = (jnp.dot(h1, wm0_ref[...], preferred_element_type=jnp.float32)
          + jnp.dot(tx2, wm1_ref[...], preferred_element_type=jnp.float32)
          + bm_ref[...])
    ls = (jnp.dot(h1, wl0_ref[...], preferred_element_type=jnp.float32)
          + jnp.dot(tx2, wl1_ref[...], preferred_element_type=jnp.float32)
          + bl_ref[...])
    ls = jnp.minimum(ls, MAX_LOGSTD)
    els = jnp.exp(ls)
    z = mu + eps_ref[...] * els
    z_ref[...] = z
    z128_ref[:, :ZD] = z
    z128_ref[:, ZD:] = jnp.zeros_like(z128_ref[:, ZD:])
    tot = jnp.sum(1.0 + 2.0 * ls - mu * mu - els * els)
    kl_ref[...] = jnp.reshape((-0.5 / (N * N)) * tot, (1, 1))


def _heads(h1, q0, q1, dinv, Wm0, Wm1, bm, Wl0, Wl1, bl, eps):
    HO = 2 * ZD
    full = lambda s: pl.BlockSpec(s, lambda: tuple(0 for _ in s))
    return pl.pallas_call(
        _heads_body,
        in_specs=[
            full((N, HO)), full((N, HO)), full((N, HO)), full((N, 1)),
            full((HO, ZD)), full((HO, ZD)), full((1, ZD)),
            full((HO, ZD)), full((HO, ZD)), full((1, ZD)),
            full((N, ZD)),
        ],
        out_specs=[full((N, ZD)), full((N, HID)), full((1, 1))],
        out_shape=[
            jax.ShapeDtypeStruct((N, ZD), jnp.float32),
            jax.ShapeDtypeStruct((N, HID), jnp.float32),
            jax.ShapeDtypeStruct((1, 1), jnp.float32),
        ],
    )(h1, q0, q1, dinv, Wm0, Wm1, bm, Wl0, Wl1, bl, eps)


def _loss_body(qp_ref, qn_ref, kl_ref, loss_ref):
    i = pl.program_id(0)
    sp = jnp.sum(qp_ref[...], axis=1)
    sn = jnp.sum(qn_ref[...], axis=1)
    lp = jnp.log(jnp.maximum(jax.nn.sigmoid(sp) + EPS, EPS))
    ln = jnp.log(jnp.maximum(1.0 - jax.nn.sigmoid(sn) + EPS, EPS))
    part = jnp.reshape(-(jnp.sum(lp) + jnp.sum(ln)) / E, (1, 1))

    @pl.when(i == 0)
    def _():
        loss_ref[...] = kl_ref[...]

    loss_ref[...] += part


def _loss(qp, qn, kl):
    BE = 8000
    return pl.pallas_call(
        _loss_body,
        grid=(E // BE,),
        in_specs=[
            pl.BlockSpec((BE, L), lambda i: (i, 0)),
            pl.BlockSpec((BE, L), lambda i: (i, 0)),
            pl.BlockSpec((1, 1), lambda i: (0, 0)),
        ],
        out_specs=pl.BlockSpec((1, 1), lambda i: (0, 0)),
        out_shape=jax.ShapeDtypeStruct((1, 1), jnp.float32),
        compiler_params=pltpu.CompilerParams(
            dimension_semantics=("arbitrary",)),
    )(qp, qn, kl)


@jax.jit
def kernel(x_adj, edge_index, neg_edge_index, Wt, bt, W10, W11, b1,
           Wm0, Wm1, bm, Wl0, Wl1, bl):
    row = edge_index[0]
    col = edge_index[1]

    deg_part, remap = _edge_prep(row, col)
    degT = deg_part[:, :N, 0].T

    h, g, dinv = _big(x_adj, Wt, bt.reshape(1, HID), degT)

    zpad = jnp.zeros((8, HID), jnp.float32)
    tx1_part = _scatter(jnp.concatenate([g, zpad], axis=0), remap, col)

    h1, g1 = _mid(h, tx1_part[0], tx1_part[1], dinv,
                  W10, W11, b1.reshape(1, 2 * ZD))

    tx2_part = _scatter(jnp.concatenate([g1, zpad], axis=0), remap, col)

    eps = jax.random.normal(jax.random.key(12345), (N, ZD), jnp.float32)
    z, z128, kl = _heads(h1, tx2_part[0, :, :2 * ZD], tx2_part[1, :, :2 * ZD],
                         dinv, Wm0, Wm1, bm.reshape(1, ZD),
                         Wl0, Wl1, bl.reshape(1, ZD), eps)

    qp, qn = _recon(z128, edge_index[0], edge_index[1],
                    neg_edge_index[0], neg_edge_index[1])
    loss = _loss(qp, qn, kl)
    return z, loss[0, 0]


# speedup vs baseline: 5.3071x; 1.1831x over previous
import functools
import jax
import jax.numpy as jnp
from jax import lax
from jax.experimental import pallas as pl
from jax.experimental.pallas import tpu as pltpu
from jax.experimental.pallas import tpu_sc as plsc

N = 10000
E = 320000
HID = 128
ZD = 32
EPS = 1e-15
MAX_LOGSTD = 10.0

NC, NS, L = 2, 16, 16
NW = NC * NS
EPW = E // NW
CH = 80
NCH = EPW // CH
PADROW = N
NPAD_A = 10112
RB = 632
RB_LAST = N - (NS - 1) * RB


@functools.cache
def _sc_mesh():
    return plsc.VectorSubcoreMesh(core_axis_name="c", subcore_axis_name="s",
                                  num_cores=NC, num_subcores=NS)


def _fill_zeros8(zb):
    for i in range(8):
        for c in range(8):
            zb[i, pl.ds(c * L, L)] = jnp.zeros((L,), jnp.float32)


def _zero_rows(zb, acc, start, nrows):
    def cp(i, _):
        off = pl.multiple_of(start + i * 8, 8)
        pltpu.sync_copy(zb, acc.at[pl.ds(off, 8)])
        return _
    lax.fori_loop(0, nrows // 8, cp, None)


def _edge_prep_body(row_hbm, col_hbm, deg_hbm, remap_hbm,
                    rowv, colv, rmv, ones, zb, dacc, sem):
    cid = lax.axis_index("c")
    sid = lax.axis_index("s")
    wid = sid * NC + cid
    RBA = NPAD_A // NS

    _fill_zeros8(zb)
    _zero_rows(zb, dacc, sid * RBA, RBA)

    def of(i, _):
        for c in range(8):
            ones[i, pl.ds(c * L, L)] = jnp.full((L,), 1.0, jnp.float32)
        return _
    lax.fori_loop(0, CH, of, None)
    plsc.subcore_barrier()

    def chunk(j, _):
        base = wid * EPW + j * CH
        pltpu.sync_copy(row_hbm.at[pl.ds(base, CH)], rowv)
        pltpu.sync_copy(col_hbm.at[pl.ds(base, CH)], colv)

        def vec(k, _):
            r = rowv[pl.ds(k * L, L)]
            c = colv[pl.ds(k * L, L)]
            rmv[pl.ds(k * L, L)] = jnp.where(r == c, PADROW, r).astype(jnp.int32)
            return _
        lax.fori_loop(0, CH // L, vec, None)
        pltpu.sync_copy(ones, dacc.at[rmv], add=True)
        pltpu.sync_copy(rmv, remap_hbm.at[pl.ds(base, CH)])
        return _
    lax.fori_loop(0, NCH, chunk, None)
    plsc.subcore_barrier()
    off = pl.multiple_of(sid * RBA, 8)
    pltpu.sync_copy(dacc.at[pl.ds(off, RBA)], deg_hbm.at[cid, pl.ds(off, RBA)])


def _edge_prep(row, col):
    return pl.kernel(
        _edge_prep_body,
        out_type=(
            jax.ShapeDtypeStruct((NC, NPAD_A, HID), jnp.float32),
            jax.ShapeDtypeStruct((E,), jnp.int32),
        ),
        mesh=_sc_mesh(),
        scratch_types=[
            pltpu.VMEM((CH,), jnp.int32),
            pltpu.VMEM((CH,), jnp.int32),
            pltpu.VMEM((CH,), jnp.int32),
            pltpu.VMEM((CH, HID), jnp.float32),
            pltpu.VMEM((8, HID), jnp.float32),
            pltpu.VMEM_SHARED((NPAD_A, HID), jnp.float32),
            pltpu.SemaphoreType.DMA,
        ],
    )(row, col)


def _scatter_body(table_hbm, remap_hbm, col_hbm, out_hbm,
                  iv, cv, rows, zb, acc, sem):
    cid = lax.axis_index("c")
    sid = lax.axis_index("s")
    wid = sid * NC + cid

    _fill_zeros8(zb)

    @pl.when(sid < NS - 1)
    def _():
        _zero_rows(zb, acc, sid * RB, RB)

    @pl.when(sid == NS - 1)
    def _():
        _zero_rows(zb, acc, (NS - 1) * RB, RB_LAST)

    plsc.subcore_barrier()

    def chunk(j, _):
        base = wid * EPW + j * CH
        pltpu.sync_copy(remap_hbm.at[pl.ds(base, CH)], iv)
        pltpu.sync_copy(col_hbm.at[pl.ds(base, CH)], cv)
        pltpu.async_copy(table_hbm.at[iv], rows, sem).wait()
        pltpu.sync_copy(rows, acc.at[cv], add=True)
        return _
    lax.fori_loop(0, NCH, chunk, None)
    plsc.subcore_barrier()

    @pl.when(sid < NS - 1)
    def _():
        off = pl.multiple_of(sid * RB, 8)
        pltpu.sync_copy(acc.at[pl.ds(off, RB)], out_hbm.at[cid, pl.ds(off, RB)])

    @pl.when(sid == NS - 1)
    def _():
        off = (NS - 1) * RB
        pltpu.sync_copy(acc.at[pl.ds(off, RB_LAST)],
                        out_hbm.at[cid, pl.ds(off, RB_LAST)])


def _scatter(table_pad, remap, col):
    return pl.kernel(
        _scatter_body,
        out_type=jax.ShapeDtypeStruct((NC, N, HID), jnp.float32),
        mesh=_sc_mesh(),
        scratch_types=[
            pltpu.VMEM((CH,), jnp.int32),
            pltpu.VMEM((CH,), jnp.int32),
            pltpu.VMEM((CH, HID), jnp.float32),
            pltpu.VMEM((8, HID), jnp.float32),
            pltpu.VMEM_SHARED((N, HID), jnp.float32),
            pltpu.SemaphoreType.DMA,
        ],
    )(table_pad, remap, col)


def _recon_body(z_hbm, e0_hbm, e1_hbm, n0_hbm, n1_hbm, qp_hbm, qn_hbm,
                i0a, i1a, zaa, zba, qva, i0b, i1b, zab, zbb, qvb,
                sia, sib, sga, sgb, sqa, sqb):
    cid = lax.axis_index("c")
    sid = lax.axis_index("s")
    wid = sid * NC + cid
    i0 = (i0a, i0b); i1 = (i1a, i1b)
    za = (zaa, zab); zb = (zba, zbb); qv = (qva, qvb)
    si = (sia, sib); sg = (sga, sgb); sq = (sqa, sqb)

    def run_side(idx0_hbm, idx1_hbm, q_hbm):
        def start_idx(j, b):
            base = wid * EPW + j * CH
            pltpu.async_copy(idx0_hbm.at[pl.ds(base, CH)], i0[b], si[b])
            pltpu.async_copy(idx1_hbm.at[pl.ds(base, CH)], i1[b], si[b])

        def unit(j, b, nb, first, last):
            pltpu.make_async_copy(idx0_hbm.at[pl.ds(0, CH)], i0[b], si[b]).wait()
            pltpu.make_async_copy(idx1_hbm.at[pl.ds(0, CH)], i1[b], si[b]).wait()
            ga = pltpu.async_copy(z_hbm.at[i0[b]], za[b], sg[b])
            gb = pltpu.async_copy(z_hbm.at[i1[b]], zb[b], sg[b])
            if not first:
                pltpu.make_async_copy(q_hbm.at[pl.ds(0, CH)], qv[nb], sq[nb]).wait()
            if not last:
                start_idx_next = j + 1
                @pl.when(start_idx_next < NCH)
                def _():
                    start_idx(start_idx_next, nb)
            ga.wait()
            gb.wait()
            for e in range(CH):
                a0 = za[b][e, pl.ds(0, L)]
                a1 = za[b][e, pl.ds(L, L)]
                c0 = zb[b][e, pl.ds(0, L)]
                c1 = zb[b][e, pl.ds(L, L)]
                qv[b][e, pl.ds(0, L)] = a0 * c0 + a1 * c1
            base = wid * EPW + j * CH
            pltpu.async_copy(qv[b], q_hbm.at[pl.ds(base, CH)], sq[b])

        start_idx(0, 0)
        unit(0, 0, 1, first=True, last=False)
        unit(1, 1, 0, first=False, last=False)

        def pair2(g, _):
            j = 2 + g * 2
            unit(j, 0, 1, first=False, last=False)
            unit(j + 1, 1, 0, first=False, last=False)
            return _
        lax.fori_loop(0, (NCH - 3) // 2, pair2, None)
        unit(NCH - 1, 0, 1, first=False, last=True)
        pltpu.make_async_copy(q_hbm.at[pl.ds(0, CH)], qv[0], sq[0]).wait()

    run_side(e0_hbm, e1_hbm, qp_hbm)
    run_side(n0_hbm, n1_hbm, qn_hbm)


def _recon(z128, e0, e1, n0, n1):
    buf = lambda: [pltpu.VMEM((CH,), jnp.int32), pltpu.VMEM((CH,), jnp.int32),
                   pltpu.VMEM((CH, HID), jnp.float32), pltpu.VMEM((CH, HID), jnp.float32),
                   pltpu.VMEM((CH, L), jnp.float32)]
    return pl.kernel(
        _recon_body,
        out_type=(
            jax.ShapeDtypeStruct((E, L), jnp.float32),
            jax.ShapeDtypeStruct((E, L), jnp.float32),
        ),
        mesh=_sc_mesh(),
        scratch_types=buf() + buf() + [pltpu.SemaphoreType.DMA] * 6,
    )(z128, e0, e1, n0, n1)


def _big_body(x_ref, wt_ref, bt_ref, degt_ref, h_ref, g_ref, dinv_ref):
    acc = jax.lax.dot_general(
        x_ref[...], wt_ref[...], (((1,), (0,)), ((), ())),
        preferred_element_type=jnp.float32)
    deg = jnp.sum(degt_ref[...], axis=1, keepdims=True)
    pos = deg > 0
    dinv = jnp.where(pos, lax.rsqrt(jnp.where(pos, deg, 1.0)), 0.0)
    h = jax.nn.relu(acc + bt_ref[...])
    h_ref[...] = h
    g_ref[...] = h * dinv
    dinv_ref[...] = dinv


def _big(x_adj, Wt, bt, degT):
    BM = 400
    return pl.pallas_call(
        _big_body,
        grid=(N // BM,),
        in_specs=[
            pl.BlockSpec((BM, N), lambda i: (i, 0)),
            pl.BlockSpec((N, HID), lambda i: (0, 0)),
            pl.BlockSpec((1, HID), lambda i: (0, 0)),
            pl.BlockSpec((BM, NC), lambda i: (i, 0)),
        ],
        out_specs=[
            pl.BlockSpec((BM, HID), lambda i: (i, 0)),
            pl.BlockSpec((BM, HID), lambda i: (i, 0)),
            pl.BlockSpec((BM, 1), lambda i: (i, 0)),
        ],
        out_shape=[
            jax.ShapeDtypeStruct((N, HID), jnp.float32),
            jax.ShapeDtypeStruct((N, HID), jnp.float32),
            jax.ShapeDtypeStruct((N, 1), jnp.float32),
        ],
        compiler_params=pltpu.CompilerParams(
            dimension_semantics=("arbitrary",)),
    )(x_adj, Wt, bt, degT)


def _mid_body(h_ref, p0_ref, p1_ref, dinv_ref, w10_ref, w11_ref, b1_ref,
              h1_ref, g1_ref):
    HO = 2 * ZD
    dinv = dinv_ref[...]
    tx1 = -dinv * (p0_ref[...] + p1_ref[...])
    h1 = jnp.dot(h_ref[...], w10_ref[...], preferred_element_type=jnp.float32)
    h1 = h1 + jnp.dot(tx1, w11_ref[...], preferred_element_type=jnp.float32)
    h1 = jax.nn.relu(h1 + b1_ref[...])
    h1_ref[...] = h1
    g1_ref[:, :HO] = h1 * dinv
    g1_ref[:, HO:] = jnp.zeros_like(g1_ref[:, HO:])


def _mid(h, p0, p1, dinv, W10, W11, b1):
    BM = 1000
    HO = 2 * ZD
    return pl.pallas_call(
        _mid_body,
        grid=(N // BM,),
        in_specs=[
            pl.BlockSpec((BM, HID), lambda i: (i, 0)),
            pl.BlockSpec((BM, HID), lambda i: (i, 0)),
            pl.BlockSpec((BM, HID), lambda i: (i, 0)),
            pl.BlockSpec((BM, 1), lambda i: (i, 0)),
            pl.BlockSpec((HID, HO), lambda i: (0, 0)),
            pl.BlockSpec((HID, HO), lambda i: (0, 0)),
            pl.BlockSpec((1, HO), lambda i: (0, 0)),
        ],
        out_specs=[
            pl.BlockSpec((BM, HO), lambda i: (i, 0)),
            pl.BlockSpec((BM, HID), lambda i: (i, 0)),
        ],
        out_shape=[
            jax.ShapeDtypeStruct((N, HO), jnp.float32),
            jax.ShapeDtypeStruct((N, HID), jnp.float32),
        ],
        compiler_params=pltpu.CompilerParams(
            dimension_semantics=("arbitrary",)),
    )(h, p0, p1, dinv, W10, W11, b1)


def _heads_body(h1_ref, q0_ref, q1_ref, dinv_ref, wm0_ref, wm1_ref, bm_ref,
                wl0_ref, wl1_ref, bl_ref, eps_ref, z_ref, kl_ref):
    tx2 = -dinv_ref[...] * (q0_ref[...] + q1_ref[...])
    h1 = h1_ref[...]
    mu = (jnp.dot(h1, wm0_ref[...], preferred_element_type=jnp.float32)
          + jnp.dot(tx2, wm1_ref[...], preferred_element_type=jnp.float32)
          + bm_ref[...])
    ls = (jnp.dot(h1, wl0_ref[...], preferred_element_type=jnp.float32)
          + jnp.dot(tx2, wl1_ref[...], preferred_element_type=jnp.float32)
          + bl_ref[...])
    ls = jnp.minimum(ls, MAX_LOGSTD)
    els = jnp.exp(ls)
    z = mu + eps_ref[...] * els
    z_ref[...] = z
    tot = jnp.sum(1.0 + 2.0 * ls - mu * mu - els * els)
    kl_ref[...] = jnp.reshape((-0.5 / (N * N)) * tot, (1, 1))


def _heads(h1, q0, q1, dinv, Wm0, Wm1, bm, Wl0, Wl1, bl, eps):
    HO = 2 * ZD
    full = lambda s: pl.BlockSpec(s, lambda: tuple(0 for _ in s))
    return pl.pallas_call(
        _heads_body,
        in_specs=[
            full((N, HO)), full((N, HO)), full((N, HO)), full((N, 1)),
            full((HO, ZD)), full((HO, ZD)), full((1, ZD)),
            full((HO, ZD)), full((HO, ZD)), full((1, ZD)),
            full((N, ZD)),
        ],
        out_specs=[full((N, ZD)), full((1, 1))],
        out_shape=[
            jax.ShapeDtypeStruct((N, ZD), jnp.float32),
            jax.ShapeDtypeStruct((1, 1), jnp.float32),
        ],
    )(h1, q0, q1, dinv, Wm0, Wm1, bm, Wl0, Wl1, bl, eps)


def _loss_body(qp_ref, qn_ref, kl_ref, loss_ref):
    i = pl.program_id(0)
    sp = jnp.sum(qp_ref[...], axis=1)
    sn = jnp.sum(qn_ref[...], axis=1)
    lp = jnp.log(jnp.maximum(jax.nn.sigmoid(sp) + EPS, EPS))
    ln = jnp.log(jnp.maximum(1.0 - jax.nn.sigmoid(sn) + EPS, EPS))
    part = jnp.reshape(-(jnp.sum(lp) + jnp.sum(ln)) / E, (1, 1))

    @pl.when(i == 0)
    def _():
        loss_ref[...] = kl_ref[...]

    loss_ref[...] += part


def _loss(qp, qn, kl):
    BE = 8000
    return pl.pallas_call(
        _loss_body,
        grid=(E // BE,),
        in_specs=[
            pl.BlockSpec((BE, L), lambda i: (i, 0)),
            pl.BlockSpec((BE, L), lambda i: (i, 0)),
            pl.BlockSpec((1, 1), lambda i: (0, 0)),
        ],
        out_specs=pl.BlockSpec((1, 1), lambda i: (0, 0)),
        out_shape=jax.ShapeDtypeStruct((1, 1), jnp.float32),
        compiler_params=pltpu.CompilerParams(
            dimension_semantics=("arbitrary",)),
    )(qp, qn, kl)


@jax.jit
def kernel(x_adj, edge_index, neg_edge_index, Wt, bt, W10, W11, b1,
           Wm0, Wm1, bm, Wl0, Wl1, bl):
    row = edge_index[0]
    col = edge_index[1]

    deg_part, remap = _edge_prep(row, col)
    degT = deg_part[:, :N, 0].T

    h, g, dinv = _big(x_adj, Wt, bt.reshape(1, HID), degT)

    zpad = jnp.zeros((8, HID), jnp.float32)
    tx1_part = _scatter(jnp.concatenate([g, zpad], axis=0), remap, col)

    h1, g1 = _mid(h, tx1_part[0], tx1_part[1], dinv,
                  W10, W11, b1.reshape(1, 2 * ZD))

    tx2_part = _scatter(jnp.concatenate([g1, zpad], axis=0), remap, col)

    eps = jax.random.normal(jax.random.key(12345), (N, ZD), jnp.float32)
    z, kl = _heads(h1, tx2_part[0, :, :2 * ZD], tx2_part[1, :, :2 * ZD],
                   dinv, Wm0, Wm1, bm.reshape(1, ZD),
                   Wl0, Wl1, bl.reshape(1, ZD), eps)

    z128 = jnp.concatenate([z, jnp.zeros((N, HID - ZD), jnp.float32)], axis=1)
    qp, qn = _recon(z128, edge_index[0], edge_index[1],
                    neg_edge_index[0], neg_edge_index[1])
    loss = _loss(qp, qn, kl)
    return z, loss[0, 0]


# speedup vs baseline: 6.7866x; 1.2788x over previous
import functools
import jax
import jax.numpy as jnp
from jax import lax
from jax.experimental import pallas as pl
from jax.experimental.pallas import tpu as pltpu
from jax.experimental.pallas import tpu_sc as plsc

N = 10000
E = 320000
HID = 128
ZD = 32
EPS = 1e-15
MAX_LOGSTD = 10.0

NC, NS, L = 2, 16, 16
NW = NC * NS
EPW = E // NW
CH = 80
NCH = EPW // CH
PADROW = N
NPAD_A = 10112
RB = 632
RB_LAST = N - (NS - 1) * RB


@functools.cache
def _sc_mesh():
    return plsc.VectorSubcoreMesh(core_axis_name="c", subcore_axis_name="s",
                                  num_cores=NC, num_subcores=NS)


def _fill_zeros8(zb):
    for i in range(8):
        for c in range(8):
            zb[i, pl.ds(c * L, L)] = jnp.zeros((L,), jnp.float32)


def _zero_rows(zb, acc, start, nrows):
    def cp(i, _):
        off = pl.multiple_of(start + i * 8, 8)
        pltpu.sync_copy(zb, acc.at[pl.ds(off, 8)])
        return _
    lax.fori_loop(0, nrows // 8, cp, None)


def _edge_prep_body(row_hbm, col_hbm, deg_hbm, remap_hbm,
                    rowv, colv, rmv, ones, zb, dacc, sem):
    cid = lax.axis_index("c")
    sid = lax.axis_index("s")
    wid = sid * NC + cid
    RBA = NPAD_A // NS

    _fill_zeros8(zb)
    _zero_rows(zb, dacc, sid * RBA, RBA)

    def of(i, _):
        for c in range(8):
            ones[i, pl.ds(c * L, L)] = jnp.full((L,), 1.0, jnp.float32)
        return _
    lax.fori_loop(0, CH, of, None)
    plsc.subcore_barrier()

    def chunk(j, _):
        base = wid * EPW + j * CH
        pltpu.sync_copy(row_hbm.at[pl.ds(base, CH)], rowv)
        pltpu.sync_copy(col_hbm.at[pl.ds(base, CH)], colv)

        def vec(k, _):
            r = rowv[pl.ds(k * L, L)]
            c = colv[pl.ds(k * L, L)]
            rmv[pl.ds(k * L, L)] = jnp.where(r == c, PADROW, r).astype(jnp.int32)
            return _
        lax.fori_loop(0, CH // L, vec, None)
        pltpu.sync_copy(ones, dacc.at[rmv], add=True)
        pltpu.sync_copy(rmv, remap_hbm.at[pl.ds(base, CH)])
        return _
    lax.fori_loop(0, NCH, chunk, None)
    plsc.subcore_barrier()
    off = pl.multiple_of(sid * RBA, 8)
    pltpu.sync_copy(dacc.at[pl.ds(off, RBA)], deg_hbm.at[cid, pl.ds(off, RBA)])


def _edge_prep(row, col):
    return pl.kernel(
        _edge_prep_body,
        out_type=(
            jax.ShapeDtypeStruct((NC, NPAD_A, HID), jnp.float32),
            jax.ShapeDtypeStruct((E,), jnp.int32),
        ),
        mesh=_sc_mesh(),
        scratch_types=[
            pltpu.VMEM((CH,), jnp.int32),
            pltpu.VMEM((CH,), jnp.int32),
            pltpu.VMEM((CH,), jnp.int32),
            pltpu.VMEM((CH, HID), jnp.float32),
            pltpu.VMEM((8, HID), jnp.float32),
            pltpu.VMEM_SHARED((NPAD_A, HID), jnp.float32),
            pltpu.SemaphoreType.DMA,
        ],
    )(row, col)


def _scatter_body(table_hbm, remap_hbm, col_hbm, out_hbm,
                  iv0, cv0, rows0, iv1, cv1, rows1, iv2, cv2, rows2,
                  zb, acc, si0, si1, si2, sg0, sg1, sg2, ss0, ss1, ss2):
    cid = lax.axis_index("c")
    sid = lax.axis_index("s")
    wid = sid * NC + cid
    iv = (iv0, iv1, iv2); cv = (cv0, cv1, cv2); rows = (rows0, rows1, rows2)
    si = (si0, si1, si2); sg = (sg0, sg1, sg2); ss = (ss0, ss1, ss2)

    _fill_zeros8(zb)

    @pl.when(sid < NS - 1)
    def _():
        _zero_rows(zb, acc, sid * RB, RB)

    @pl.when(sid == NS - 1)
    def _():
        _zero_rows(zb, acc, (NS - 1) * RB, RB_LAST)

    plsc.subcore_barrier()

    def start_idx(j, b):
        base = wid * EPW + j * CH
        pltpu.async_copy(remap_hbm.at[pl.ds(base, CH)], iv[b], si[b])
        pltpu.async_copy(col_hbm.at[pl.ds(base, CH)], cv[b], si[b])

    def wait_idx(b):
        pltpu.make_async_copy(remap_hbm.at[pl.ds(0, CH)], iv[b], si[b]).wait()
        pltpu.make_async_copy(col_hbm.at[pl.ds(0, CH)], cv[b], si[b]).wait()

    def wait_gather(b):
        pltpu.make_async_copy(table_hbm.at[iv[b]], rows[b], sg[b]).wait()

    def wait_scatter(b):
        pltpu.make_async_copy(rows[b], acc.at[cv[b]], ss[b]).wait()

    def unit(j, b, bp, bm, guard):
        wait_gather(b)
        wait_scatter(bm)
        if guard:
            @pl.when(j + 2 < NCH)
            def _():
                start_idx(j + 2, bm)
            @pl.when(j + 1 < NCH)
            def _():
                wait_idx(bp)
                pltpu.async_copy(table_hbm.at[iv[bp]], rows[bp], sg[bp])
        else:
            start_idx(j + 2, bm)
            wait_idx(bp)
            pltpu.async_copy(table_hbm.at[iv[bp]], rows[bp], sg[bp])
        pltpu.async_copy(rows[b], acc.at[cv[b]], ss[b], add=True)

    start_idx(0, 0)
    wait_idx(0)
    pltpu.async_copy(table_hbm.at[iv[0]], rows[0], sg[0])
    start_idx(1, 1)
    wait_gather(0)
    start_idx(2, 2)
    wait_idx(1)
    pltpu.async_copy(table_hbm.at[iv[1]], rows[1], sg[1])
    pltpu.async_copy(rows[0], acc.at[cv[0]], ss[0], add=True)
    wait_gather(1)
    wait_scatter(0)
    start_idx(3, 0)
    wait_idx(2)
    pltpu.async_copy(table_hbm.at[iv[2]], rows[2], sg[2])
    pltpu.async_copy(rows[1], acc.at[cv[1]], ss[1], add=True)

    def triple(g, _):
        j = 2 + 3 * g
        unit(j, 2, 0, 1, guard=(True))
        unit(j + 1, 0, 1, 2, guard=True)
        unit(j + 2, 1, 2, 0, guard=True)
        return _
    lax.fori_loop(0, (NCH - 2) // 3, triple, None)
    wait_scatter((NCH - 1) % 3)
    plsc.subcore_barrier()

    @pl.when(sid < NS - 1)
    def _():
        off = pl.multiple_of(sid * RB, 8)
        pltpu.sync_copy(acc.at[pl.ds(off, RB)], out_hbm.at[cid, pl.ds(off, RB)])

    @pl.when(sid == NS - 1)
    def _():
        off = (NS - 1) * RB
        pltpu.sync_copy(acc.at[pl.ds(off, RB_LAST)],
                        out_hbm.at[cid, pl.ds(off, RB_LAST)])


def _scatter(table_pad, remap, col):
    buf = lambda: [pltpu.VMEM((CH,), jnp.int32), pltpu.VMEM((CH,), jnp.int32),
                   pltpu.VMEM((CH, HID), jnp.float32)]
    return pl.kernel(
        _scatter_body,
        out_type=jax.ShapeDtypeStruct((NC, N, HID), jnp.float32),
        mesh=_sc_mesh(),
        scratch_types=buf() + buf() + buf() + [
            pltpu.VMEM((8, HID), jnp.float32),
            pltpu.VMEM_SHARED((N, HID), jnp.float32),
        ] + [pltpu.SemaphoreType.DMA] * 9,
    )(table_pad, remap, col)


def _recon_body(z_hbm, e0_hbm, e1_hbm, n0_hbm, n1_hbm, qp_hbm, qn_hbm,
                i0a, i1a, zaa, zba, qva, i0b, i1b, zab, zbb, qvb,
                i0c, i1c, zac, zbc, qvc,
                sia, sib, sic, sga, sgb, sgc, sqa, sqb, sqc):
    cid = lax.axis_index("c")
    sid = lax.axis_index("s")
    wid = sid * NC + cid
    i0 = (i0a, i0b, i0c); i1 = (i1a, i1b, i1c)
    za = (zaa, zab, zac); zb = (zba, zbb, zbc); qv = (qva, qvb, qvc)
    si = (sia, sib, sic); sg = (sga, sgb, sgc); sq = (sqa, sqb, sqc)

    def run_side(idx0_hbm, idx1_hbm, q_hbm):
        def start_idx(j, b):
            base = wid * EPW + j * CH
            pltpu.async_copy(idx0_hbm.at[pl.ds(base, CH)], i0[b], si[b])
            pltpu.async_copy(idx1_hbm.at[pl.ds(base, CH)], i1[b], si[b])

        def wait_idx(b):
            pltpu.make_async_copy(idx0_hbm.at[pl.ds(0, CH)], i0[b], si[b]).wait()
            pltpu.make_async_copy(idx1_hbm.at[pl.ds(0, CH)], i1[b], si[b]).wait()

        def start_gathers(b):
            pltpu.async_copy(z_hbm.at[i0[b]], za[b], sg[b])
            pltpu.async_copy(z_hbm.at[i1[b]], zb[b], sg[b])

        def wait_gathers(b):
            pltpu.make_async_copy(z_hbm.at[i0[b]], za[b], sg[b]).wait()
            pltpu.make_async_copy(z_hbm.at[i1[b]], zb[b], sg[b]).wait()

        def wait_qstore(b):
            pltpu.make_async_copy(q_hbm.at[pl.ds(0, CH)], qv[b], sq[b]).wait()

        def compute(j, b):
            for e in range(CH):
                a0 = za[b][e, pl.ds(0, L)]
                a1 = za[b][e, pl.ds(L, L)]
                c0 = zb[b][e, pl.ds(0, L)]
                c1 = zb[b][e, pl.ds(L, L)]
                qv[b][e, pl.ds(0, L)] = a0 * c0 + a1 * c1
            base = wid * EPW + j * CH
            pltpu.async_copy(qv[b], q_hbm.at[pl.ds(base, CH)], sq[b])

        def unit(j, b, bp, bm, guard):
            wait_gathers(b)
            wait_qstore(bm)
            if guard:
                @pl.when(j + 2 < NCH)
                def _():
                    start_idx(j + 2, bm)
                @pl.when(j + 1 < NCH)
                def _():
                    wait_idx(bp)
                    start_gathers(bp)
            else:
                start_idx(j + 2, bm)
                wait_idx(bp)
                start_gathers(bp)
            compute(j, b)

        start_idx(0, 0)
        wait_idx(0)
        start_gathers(0)
        start_idx(1, 1)
        wait_gathers(0)
        start_idx(2, 2)
        wait_idx(1)
        start_gathers(1)
        compute(0, 0)
        wait_gathers(1)
        wait_qstore(0)
        start_idx(3, 0)
        wait_idx(2)
        start_gathers(2)
        compute(1, 1)

        def triple(g, _):
            j = 2 + 3 * g
            unit(j, 2, 0, 1, guard=(True))
            unit(j + 1, 0, 1, 2, guard=True)
            unit(j + 2, 1, 2, 0, guard=True)
            return _
        lax.fori_loop(0, (NCH - 2) // 3, triple, None)
        wait_qstore((NCH - 1) % 3)

    run_side(e0_hbm, e1_hbm, qp_hbm)
    run_side(n0_hbm, n1_hbm, qn_hbm)


def _recon(z128, e0, e1, n0, n1):
    buf = lambda: [pltpu.VMEM((CH,), jnp.int32), pltpu.VMEM((CH,), jnp.int32),
                   pltpu.VMEM((CH, HID), jnp.float32), pltpu.VMEM((CH, HID), jnp.float32),
                   pltpu.VMEM((CH, L), jnp.float32)]
    return pl.kernel(
        _recon_body,
        out_type=(
            jax.ShapeDtypeStruct((E, L), jnp.float32),
            jax.ShapeDtypeStruct((E, L), jnp.float32),
        ),
        mesh=_sc_mesh(),
        scratch_types=buf() + buf() + buf() + [pltpu.SemaphoreType.DMA] * 9,
    )(z128, e0, e1, n0, n1)


def _big_body(x_ref, wt_ref, bt_ref, degt_ref, h_ref, g_ref, dinv_ref):
    acc = jax.lax.dot_general(
        x_ref[...], wt_ref[...], (((1,), (0,)), ((), ())),
        preferred_element_type=jnp.float32)
    deg = jnp.sum(degt_ref[...], axis=1, keepdims=True)
    pos = deg > 0
    dinv = jnp.where(pos, lax.rsqrt(jnp.where(pos, deg, 1.0)), 0.0)
    h = jax.nn.relu(acc + bt_ref[...])
    h_ref[...] = h
    g_ref[...] = h * dinv
    dinv_ref[...] = dinv


def _big(x_adj, Wt, bt, degT):
    BM = 400
    return pl.pallas_call(
        _big_body,
        grid=(N // BM,),
        in_specs=[
            pl.BlockSpec((BM, N), lambda i: (i, 0)),
            pl.BlockSpec((N, HID), lambda i: (0, 0)),
            pl.BlockSpec((1, HID), lambda i: (0, 0)),
            pl.BlockSpec((BM, NC), lambda i: (i, 0)),
        ],
        out_specs=[
            pl.BlockSpec((BM, HID), lambda i: (i, 0)),
            pl.BlockSpec((BM, HID), lambda i: (i, 0)),
            pl.BlockSpec((BM, 1), lambda i: (i, 0)),
        ],
        out_shape=[
            jax.ShapeDtypeStruct((N, HID), jnp.float32),
            jax.ShapeDtypeStruct((N, HID), jnp.float32),
            jax.ShapeDtypeStruct((N, 1), jnp.float32),
        ],
        compiler_params=pltpu.CompilerParams(
            dimension_semantics=("arbitrary",)),
    )(x_adj, Wt, bt, degT)


def _mid_body(h_ref, p0_ref, p1_ref, dinv_ref, w10_ref, w11_ref, b1_ref,
              h1_ref, g1_ref):
    HO = 2 * ZD
    dinv = dinv_ref[...]
    tx1 = -dinv * (p0_ref[...] + p1_ref[...])
    h1 = jnp.dot(h_ref[...], w10_ref[...], preferred_element_type=jnp.float32)
    h1 = h1 + jnp.dot(tx1, w11_ref[...], preferred_element_type=jnp.float32)
    h1 = jax.nn.relu(h1 + b1_ref[...])
    h1_ref[...] = h1
    g1_ref[:, :HO] = h1 * dinv
    g1_ref[:, HO:] = jnp.zeros_like(g1_ref[:, HO:])


def _mid(h, p0, p1, dinv, W10, W11, b1):
    BM = 1000
    HO = 2 * ZD
    return pl.pallas_call(
        _mid_body,
        grid=(N // BM,),
        in_specs=[
            pl.BlockSpec((BM, HID), lambda i: (i, 0)),
            pl.BlockSpec((BM, HID), lambda i: (i, 0)),
            pl.BlockSpec((BM, HID), lambda i: (i, 0)),
            pl.BlockSpec((BM, 1), lambda i: (i, 0)),
            pl.BlockSpec((HID, HO), lambda i: (0, 0)),
            pl.BlockSpec((HID, HO), lambda i: (0, 0)),
            pl.BlockSpec((1, HO), lambda i: (0, 0)),
        ],
        out_specs=[
            pl.BlockSpec((BM, HO), lambda i: (i, 0)),
            pl.BlockSpec((BM, HID), lambda i: (i, 0)),
        ],
        out_shape=[
            jax.ShapeDtypeStruct((N, HO), jnp.float32),
            jax.ShapeDtypeStruct((N, HID), jnp.float32),
        ],
        compiler_params=pltpu.CompilerParams(
            dimension_semantics=("arbitrary",)),
    )(h, p0, p1, dinv, W10, W11, b1)


def _heads_body(h1_ref, q0_ref, q1_ref, dinv_ref, wm0_ref, wm1_ref, bm_ref,
                wl0_ref, wl1_ref, bl_ref, eps_ref, z_ref, kl_ref):
    tx2 = -dinv_ref[...] * (q0_ref[...] + q1_ref[...])
    h1 = h1_ref[...]
    mu = (jnp.dot(h1, wm0_ref[...], preferred_element_type=jnp.float32)
          + jnp.dot(tx2, wm1_ref[...], preferred_element_type=jnp.float32)
          + bm_ref[...])
    ls = (jnp.dot(h1, wl0_ref[...], preferred_element_type=jnp.float32)
          + jnp.dot(tx2, wl1_ref[...], preferred_element_type=jnp.float32)
          + bl_ref[...])
    ls = jnp.minimum(ls, MAX_LOGSTD)
    els = jnp.exp(ls)
    z = mu + eps_ref[...] * els
    z_ref[...] = z
    tot = jnp.sum(1.0 + 2.0 * ls - mu * mu - els * els)
    kl_ref[...] = jnp.reshape((-0.5 / (N * N)) * tot, (1, 1))


def _heads(h1, q0, q1, dinv, Wm0, Wm1, bm, Wl0, Wl1, bl, eps):
    HO = 2 * ZD
    full = lambda s: pl.BlockSpec(s, lambda: tuple(0 for _ in s))
    return pl.pallas_call(
        _heads_body,
        in_specs=[
            full((N, HO)), full((N, HO)), full((N, HO)), full((N, 1)),
            full((HO, ZD)), full((HO, ZD)), full((1, ZD)),
            full((HO, ZD)), full((HO, ZD)), full((1, ZD)),
            full((N, ZD)),
        ],
        out_specs=[full((N, ZD)), full((1, 1))],
        out_shape=[
            jax.ShapeDtypeStruct((N, ZD), jnp.float32),
            jax.ShapeDtypeStruct((1, 1), jnp.float32),
        ],
    )(h1, q0, q1, dinv, Wm0, Wm1, bm, Wl0, Wl1, bl, eps)


def _loss_body(qp_ref, qn_ref, kl_ref, loss_ref):
    i = pl.program_id(0)
    sp = jnp.sum(qp_ref[...], axis=1)
    sn = jnp.sum(qn_ref[...], axis=1)
    lp = jnp.log(jnp.maximum(jax.nn.sigmoid(sp) + EPS, EPS))
    ln = jnp.log(jnp.maximum(1.0 - jax.nn.sigmoid(sn) + EPS, EPS))
    part = jnp.reshape(-(jnp.sum(lp) + jnp.sum(ln)) / E, (1, 1))

    @pl.when(i == 0)
    def _():
        loss_ref[...] = kl_ref[...]

    loss_ref[...] += part


def _loss(qp, qn, kl):
    BE = 8000
    return pl.pallas_call(
        _loss_body,
        grid=(E // BE,),
        in_specs=[
            pl.BlockSpec((BE, L), lambda i: (i, 0)),
            pl.BlockSpec((BE, L), lambda i: (i, 0)),
            pl.BlockSpec((1, 1), lambda i: (0, 0)),
        ],
        out_specs=pl.BlockSpec((1, 1), lambda i: (0, 0)),
        out_shape=jax.ShapeDtypeStruct((1, 1), jnp.float32),
        compiler_params=pltpu.CompilerParams(
            dimension_semantics=("arbitrary",)),
    )(qp, qn, kl)


@jax.jit
def kernel(x_adj, edge_index, neg_edge_index, Wt, bt, W10, W11, b1,
           Wm0, Wm1, bm, Wl0, Wl1, bl):
    row = edge_index[0]
    col = edge_index[1]

    deg_part, remap = _edge_prep(row, col)
    degT = deg_part[:, :N, 0].T

    h, g, dinv = _big(x_adj, Wt, bt.reshape(1, HID), degT)

    zpad = jnp.zeros((8, HID), jnp.float32)
    tx1_part = _scatter(jnp.concatenate([g, zpad], axis=0), remap, col)

    h1, g1 = _mid(h, tx1_part[0], tx1_part[1], dinv,
                  W10, W11, b1.reshape(1, 2 * ZD))

    tx2_part = _scatter(jnp.concatenate([g1, zpad], axis=0), remap, col)

    eps = jax.random.normal(jax.random.key(12345), (N, ZD), jnp.float32)
    z, kl = _heads(h1, tx2_part[0, :, :2 * ZD], tx2_part[1, :, :2 * ZD],
                   dinv, Wm0, Wm1, bm.reshape(1, ZD),
                   Wl0, Wl1, bl.reshape(1, ZD), eps)

    z128 = jnp.concatenate([z, jnp.zeros((N, HID - ZD), jnp.float32)], axis=1)
    qp, qn = _recon(z128, edge_index[0], edge_index[1],
                    neg_edge_index[0], neg_edge_index[1])
    loss = _loss(qp, qn, kl)
    return z, loss[0, 0]


# speedup vs baseline: 7.1318x; 1.0509x over previous
import functools
import jax
import jax.numpy as jnp
from jax import lax
from jax.experimental import pallas as pl
from jax.experimental.pallas import tpu as pltpu
from jax.experimental.pallas import tpu_sc as plsc

N = 10000
E = 320000
HID = 128
ZD = 32
EPS = 1e-15
MAX_LOGSTD = 10.0

NC, NS, L = 2, 16, 16
NW = NC * NS
EPW = E // NW
CH = 80
NCH = EPW // CH
PADROW = N
NPAD_A = 10112
RB = 632
RB_LAST = N - (NS - 1) * RB


@functools.cache
def _sc_mesh():
    return plsc.VectorSubcoreMesh(core_axis_name="c", subcore_axis_name="s",
                                  num_cores=NC, num_subcores=NS)


def _fill_zeros8(zb):
    for i in range(8):
        for c in range(8):
            zb[i, pl.ds(c * L, L)] = jnp.zeros((L,), jnp.float32)


def _zero_rows(zb, acc, start, nrows):
    def cp(i, _):
        off = pl.multiple_of(start + i * 8, 8)
        pltpu.sync_copy(zb, acc.at[pl.ds(off, 8)])
        return _
    lax.fori_loop(0, nrows // 8, cp, None)


def _edge_prep_body(row_hbm, col_hbm, deg_hbm, remap_hbm,
                    rowv0, colv0, rmv0, rowv1, colv1, rmv1, rowv2, colv2, rmv2,
                    ones, zb, dacc,
                    si0, si1, si2, ss0, ss1, ss2, sr0, sr1, sr2):
    cid = lax.axis_index("c")
    sid = lax.axis_index("s")
    wid = sid * NC + cid
    RBA = NPAD_A // NS
    rowv = (rowv0, rowv1, rowv2)
    colv = (colv0, colv1, colv2)
    rmv = (rmv0, rmv1, rmv2)
    si = (si0, si1, si2); ss = (ss0, ss1, ss2); sr = (sr0, sr1, sr2)

    _fill_zeros8(zb)
    _zero_rows(zb, dacc, sid * RBA, RBA)

    def of(i, _):
        for c in range(8):
            ones[i, pl.ds(c * L, L)] = jnp.full((L,), 1.0, jnp.float32)
        return _
    lax.fori_loop(0, CH, of, None)
    plsc.subcore_barrier()

    def start_idx(j, b):
        base = wid * EPW + j * CH
        pltpu.async_copy(row_hbm.at[pl.ds(base, CH)], rowv[b], si[b])
        pltpu.async_copy(col_hbm.at[pl.ds(base, CH)], colv[b], si[b])

    def wait_idx(b):
        pltpu.make_async_copy(row_hbm.at[pl.ds(0, CH)], rowv[b], si[b]).wait()
        pltpu.make_async_copy(col_hbm.at[pl.ds(0, CH)], colv[b], si[b]).wait()

    def wait_scatter(b):
        pltpu.make_async_copy(ones, dacc.at[rmv[b]], ss[b]).wait()

    def wait_rstore(b):
        pltpu.make_async_copy(remap_hbm.at[pl.ds(0, CH)], rmv[b], sr[b]).wait()

    def compute(b):
        @plsc.parallel_loop(0, CH // L, unroll=5)
        def _(k):
            r = rowv[b][pl.ds(k * L, L)]
            c = colv[b][pl.ds(k * L, L)]
            rmv[b][pl.ds(k * L, L)] = jnp.where(r == c, PADROW, r).astype(jnp.int32)

    def emit(j, b):
        base = wid * EPW + j * CH
        pltpu.async_copy(ones, dacc.at[rmv[b]], ss[b], add=True)
        pltpu.async_copy(rmv[b], remap_hbm.at[pl.ds(base, CH)], sr[b])

    def unit(j, b, bp, bm, guard):
        wait_idx(b)
        wait_scatter(bm)
        wait_rstore(bm)
        if guard:
            @pl.when(j + 2 < NCH)
            def _():
                start_idx(j + 2, bm)
        else:
            start_idx(j + 2, bm)
        compute(b)
        emit(j, b)

    start_idx(0, 0)
    start_idx(1, 1)
    wait_idx(0)
    start_idx(2, 2)
    compute(0)
    emit(0, 0)
    wait_idx(1)
    wait_scatter(0)
    wait_rstore(0)
    start_idx(3, 0)
    compute(1)
    emit(1, 1)

    def triple(g, _):
        j = 2 + 3 * g
        unit(j, 2, 0, 1, guard=(True))
        unit(j + 1, 0, 1, 2, guard=True)
        unit(j + 2, 1, 2, 0, guard=True)
        return _
    lax.fori_loop(0, (NCH - 2) // 3, triple, None)
    wait_scatter((NCH - 1) % 3)
    wait_rstore((NCH - 1) % 3)

    plsc.subcore_barrier()
    off = pl.multiple_of(sid * RBA, 8)
    pltpu.sync_copy(dacc.at[pl.ds(off, RBA)], deg_hbm.at[cid, pl.ds(off, RBA)])


def _edge_prep(row, col):
    buf = lambda: [pltpu.VMEM((CH,), jnp.int32), pltpu.VMEM((CH,), jnp.int32),
                   pltpu.VMEM((CH,), jnp.int32)]
    return pl.kernel(
        _edge_prep_body,
        out_type=(
            jax.ShapeDtypeStruct((NC, NPAD_A, HID), jnp.float32),
            jax.ShapeDtypeStruct((E,), jnp.int32),
        ),
        mesh=_sc_mesh(),
        scratch_types=buf() + buf() + buf() + [
            pltpu.VMEM((CH, HID), jnp.float32),
            pltpu.VMEM((8, HID), jnp.float32),
            pltpu.VMEM_SHARED((NPAD_A, HID), jnp.float32),
        ] + [pltpu.SemaphoreType.DMA] * 9,
    )(row, col)


def _scatter_body(table_hbm, remap_hbm, col_hbm, out_hbm,
                  iv0, cv0, rows0, iv1, cv1, rows1, iv2, cv2, rows2,
                  zb, acc, si0, si1, si2, sg0, sg1, sg2, ss0, ss1, ss2):
    cid = lax.axis_index("c")
    sid = lax.axis_index("s")
    wid = sid * NC + cid
    iv = (iv0, iv1, iv2); cv = (cv0, cv1, cv2); rows = (rows0, rows1, rows2)
    si = (si0, si1, si2); sg = (sg0, sg1, sg2); ss = (ss0, ss1, ss2)

    _fill_zeros8(zb)

    @pl.when(sid < NS - 1)
    def _():
        _zero_rows(zb, acc, sid * RB, RB)

    @pl.when(sid == NS - 1)
    def _():
        _zero_rows(zb, acc, (NS - 1) * RB, RB_LAST)

    plsc.subcore_barrier()

    def start_idx(j, b):
        base = wid * EPW + j * CH
        pltpu.async_copy(remap_hbm.at[pl.ds(base, CH)], iv[b], si[b])
        pltpu.async_copy(col_hbm.at[pl.ds(base, CH)], cv[b], si[b])

    def wait_idx(b):
        pltpu.make_async_copy(remap_hbm.at[pl.ds(0, CH)], iv[b], si[b]).wait()
        pltpu.make_async_copy(col_hbm.at[pl.ds(0, CH)], cv[b], si[b]).wait()

    def wait_gather(b):
        pltpu.make_async_copy(table_hbm.at[iv[b]], rows[b], sg[b]).wait()

    def wait_scatter(b):
        pltpu.make_async_copy(rows[b], acc.at[cv[b]], ss[b]).wait()

    def unit(j, b, bp, bm, guard):
        wait_gather(b)
        wait_scatter(bm)
        if guard:
            @pl.when(j + 2 < NCH)
            def _():
                start_idx(j + 2, bm)
            @pl.when(j + 1 < NCH)
            def _():
                wait_idx(bp)
                pltpu.async_copy(table_hbm.at[iv[bp]], rows[bp], sg[bp])
        else:
            start_idx(j + 2, bm)
            wait_idx(bp)
            pltpu.async_copy(table_hbm.at[iv[bp]], rows[bp], sg[bp])
        pltpu.async_copy(rows[b], acc.at[cv[b]], ss[b], add=True)

    start_idx(0, 0)
    wait_idx(0)
    pltpu.async_copy(table_hbm.at[iv[0]], rows[0], sg[0])
    start_idx(1, 1)
    wait_gather(0)
    start_idx(2, 2)
    wait_idx(1)
    pltpu.async_copy(table_hbm.at[iv[1]], rows[1], sg[1])
    pltpu.async_copy(rows[0], acc.at[cv[0]], ss[0], add=True)
    wait_gather(1)
    wait_scatter(0)
    start_idx(3, 0)
    wait_idx(2)
    pltpu.async_copy(table_hbm.at[iv[2]], rows[2], sg[2])
    pltpu.async_copy(rows[1], acc.at[cv[1]], ss[1], add=True)

    def triple(g, _):
        j = 2 + 3 * g
        unit(j, 2, 0, 1, guard=(True))
        unit(j + 1, 0, 1, 2, guard=True)
        unit(j + 2, 1, 2, 0, guard=True)
        return _
    lax.fori_loop(0, (NCH - 2) // 3, triple, None)
    wait_scatter((NCH - 1) % 3)
    plsc.subcore_barrier()

    @pl.when(sid < NS - 1)
    def _():
        off = pl.multiple_of(sid * RB, 8)
        pltpu.sync_copy(acc.at[pl.ds(off, RB)], out_hbm.at[cid, pl.ds(off, RB)])

    @pl.when(sid == NS - 1)
    def _():
        off = (NS - 1) * RB
        pltpu.sync_copy(acc.at[pl.ds(off, RB_LAST)],
                        out_hbm.at[cid, pl.ds(off, RB_LAST)])


def _scatter(table_pad, remap, col):
    buf = lambda: [pltpu.VMEM((CH,), jnp.int32), pltpu.VMEM((CH,), jnp.int32),
                   pltpu.VMEM((CH, HID), jnp.float32)]
    return pl.kernel(
        _scatter_body,
        out_type=jax.ShapeDtypeStruct((NC, N, HID), jnp.float32),
        mesh=_sc_mesh(),
        scratch_types=buf() + buf() + buf() + [
            pltpu.VMEM((8, HID), jnp.float32),
            pltpu.VMEM_SHARED((N, HID), jnp.float32),
        ] + [pltpu.SemaphoreType.DMA] * 9,
    )(table_pad, remap, col)


def _recon_body(z_hbm, e0_hbm, e1_hbm, n0_hbm, n1_hbm, qp_hbm, qn_hbm,
                i0a, i1a, zaa, zba, qva, i0b, i1b, zab, zbb, qvb,
                i0c, i1c, zac, zbc, qvc,
                sia, sib, sic, sga, sgb, sgc, sqa, sqb, sqc):
    cid = lax.axis_index("c")
    sid = lax.axis_index("s")
    wid = sid * NC + cid
    i0 = (i0a, i0b, i0c); i1 = (i1a, i1b, i1c)
    za = (zaa, zab, zac); zb = (zba, zbb, zbc); qv = (qva, qvb, qvc)
    si = (sia, sib, sic); sg = (sga, sgb, sgc); sq = (sqa, sqb, sqc)

    def run_side(idx0_hbm, idx1_hbm, q_hbm):
        def start_idx(j, b):
            base = wid * EPW + j * CH
            pltpu.async_copy(idx0_hbm.at[pl.ds(base, CH)], i0[b], si[b])
            pltpu.async_copy(idx1_hbm.at[pl.ds(base, CH)], i1[b], si[b])

        def wait_idx(b):
            pltpu.make_async_copy(idx0_hbm.at[pl.ds(0, CH)], i0[b], si[b]).wait()
            pltpu.make_async_copy(idx1_hbm.at[pl.ds(0, CH)], i1[b], si[b]).wait()

        def start_gathers(b):
            pltpu.async_copy(z_hbm.at[i0[b]], za[b], sg[b])
            pltpu.async_copy(z_hbm.at[i1[b]], zb[b], sg[b])

        def wait_gathers(b):
            pltpu.make_async_copy(z_hbm.at[i0[b]], za[b], sg[b]).wait()
            pltpu.make_async_copy(z_hbm.at[i1[b]], zb[b], sg[b]).wait()

        def wait_qstore(b):
            pltpu.make_async_copy(q_hbm.at[pl.ds(0, CH)], qv[b], sq[b]).wait()

        def compute(j, b):
            @plsc.parallel_loop(0, CH, unroll=8)
            def _(e):
                a0 = za[b][e, pl.ds(0, L)]
                a1 = za[b][e, pl.ds(L, L)]
                c0 = zb[b][e, pl.ds(0, L)]
                c1 = zb[b][e, pl.ds(L, L)]
                qv[b][e, pl.ds(0, L)] = a0 * c0 + a1 * c1
            base = wid * EPW + j * CH
            pltpu.async_copy(qv[b], q_hbm.at[pl.ds(base, CH)], sq[b])

        def unit(j, b, bp, bm, guard):
            wait_gathers(b)
            wait_qstore(bm)
            if guard:
                @pl.when(j + 2 < NCH)
                def _():
                    start_idx(j + 2, bm)
                @pl.when(j + 1 < NCH)
                def _():
                    wait_idx(bp)
                    start_gathers(bp)
            else:
                start_idx(j + 2, bm)
                wait_idx(bp)
                start_gathers(bp)
            compute(j, b)

        start_idx(0, 0)
        wait_idx(0)
        start_gathers(0)
        start_idx(1, 1)
        wait_gathers(0)
        start_idx(2, 2)
        wait_idx(1)
        start_gathers(1)
        compute(0, 0)
        wait_gathers(1)
        wait_qstore(0)
        start_idx(3, 0)
        wait_idx(2)
        start_gathers(2)
        compute(1, 1)

        def triple(g, _):
            j = 2 + 3 * g
            unit(j, 2, 0, 1, guard=(True))
            unit(j + 1, 0, 1, 2, guard=True)
            unit(j + 2, 1, 2, 0, guard=True)
            return _
        lax.fori_loop(0, (NCH - 2) // 3, triple, None)
        wait_qstore((NCH - 1) % 3)

    run_side(e0_hbm, e1_hbm, qp_hbm)
    run_side(n0_hbm, n1_hbm, qn_hbm)


def _recon(z128, e0, e1, n0, n1):
    buf = lambda: [pltpu.VMEM((CH,), jnp.int32), pltpu.VMEM((CH,), jnp.int32),
                   pltpu.VMEM((CH, HID), jnp.float32), pltpu.VMEM((CH, HID), jnp.float32),
                   pltpu.VMEM((CH, L), jnp.float32)]
    return pl.kernel(
        _recon_body,
        out_type=(
            jax.ShapeDtypeStruct((E, L), jnp.float32),
            jax.ShapeDtypeStruct((E, L), jnp.float32),
        ),
        mesh=_sc_mesh(),
        scratch_types=buf() + buf() + buf() + [pltpu.SemaphoreType.DMA] * 9,
    )(z128, e0, e1, n0, n1)


def _big_body(x_ref, wt_ref, bt_ref, degt_ref, h_ref, g_ref, dinv_ref):
    acc = jax.lax.dot_general(
        x_ref[...], wt_ref[...], (((1,), (0,)), ((), ())),
        preferred_element_type=jnp.float32)
    deg = jnp.sum(degt_ref[...], axis=1, keepdims=True)
    pos = deg > 0
    dinv = jnp.where(pos, lax.rsqrt(jnp.where(pos, deg, 1.0)), 0.0)
    h = jax.nn.relu(acc + bt_ref[...])
    h_ref[...] = h
    g_ref[...] = h * dinv
    dinv_ref[...] = dinv


def _big(x_adj, Wt, bt, degT):
    BM = 400
    return pl.pallas_call(
        _big_body,
        grid=(N // BM,),
        in_specs=[
            pl.BlockSpec((BM, N), lambda i: (i, 0)),
            pl.BlockSpec((N, HID), lambda i: (0, 0)),
            pl.BlockSpec((1, HID), lambda i: (0, 0)),
            pl.BlockSpec((BM, NC), lambda i: (i, 0)),
        ],
        out_specs=[
            pl.BlockSpec((BM, HID), lambda i: (i, 0)),
            pl.BlockSpec((BM, HID), lambda i: (i, 0)),
            pl.BlockSpec((BM, 1), lambda i: (i, 0)),
        ],
        out_shape=[
            jax.ShapeDtypeStruct((N, HID), jnp.float32),
            jax.ShapeDtypeStruct((N, HID), jnp.float32),
            jax.ShapeDtypeStruct((N, 1), jnp.float32),
        ],
        compiler_params=pltpu.CompilerParams(
            dimension_semantics=("arbitrary",)),
    )(x_adj, Wt, bt, degT)


def _mid_body(h_ref, p0_ref, p1_ref, dinv_ref, w10_ref, w11_ref, b1_ref,
              h1_ref, g1_ref):
    HO = 2 * ZD
    dinv = dinv_ref[...]
    tx1 = -dinv * (p0_ref[...] + p1_ref[...])
    h1 = jnp.dot(h_ref[...], w10_ref[...], preferred_element_type=jnp.float32)
    h1 = h1 + jnp.dot(tx1, w11_ref[...], preferred_element_type=jnp.float32)
    h1 = jax.nn.relu(h1 + b1_ref[...])
    h1_ref[...] = h1
    g1_ref[:, :HO] = h1 * dinv
    g1_ref[:, HO:] = jnp.zeros_like(g1_ref[:, HO:])


def _mid(h, p0, p1, dinv, W10, W11, b1):
    BM = 1000
    HO = 2 * ZD
    return pl.pallas_call(
        _mid_body,
        grid=(N // BM,),
        in_specs=[
            pl.BlockSpec((BM, HID), lambda i: (i, 0)),
            pl.BlockSpec((BM, HID), lambda i: (i, 0)),
            pl.BlockSpec((BM, HID), lambda i: (i, 0)),
            pl.BlockSpec((BM, 1), lambda i: (i, 0)),
            pl.BlockSpec((HID, HO), lambda i: (0, 0)),
            pl.BlockSpec((HID, HO), lambda i: (0, 0)),
            pl.BlockSpec((1, HO), lambda i: (0, 0)),
        ],
        out_specs=[
            pl.BlockSpec((BM, HO), lambda i: (i, 0)),
            pl.BlockSpec((BM, HID), lambda i: (i, 0)),
        ],
        out_shape=[
            jax.ShapeDtypeStruct((N, HO), jnp.float32),
            jax.ShapeDtypeStruct((N, HID), jnp.float32),
        ],
        compiler_params=pltpu.CompilerParams(
            dimension_semantics=("arbitrary",)),
    )(h, p0, p1, dinv, W10, W11, b1)


def _heads_body(h1_ref, q0_ref, q1_ref, dinv_ref, wm0_ref, wm1_ref, bm_ref,
                wl0_ref, wl1_ref, bl_ref, eps_ref, z_ref, kl_ref):
    tx2 = -dinv_ref[...] * (q0_ref[...] + q1_ref[...])
    h1 = h1_ref[...]
    mu = (jnp.dot(h1, wm0_ref[...], preferred_element_type=jnp.float32)
          + jnp.dot(tx2, wm1_ref[...], preferred_element_type=jnp.float32)
          + bm_ref[...])
    ls = (jnp.dot(h1, wl0_ref[...], preferred_element_type=jnp.float32)
          + jnp.dot(tx2, wl1_ref[...], preferred_element_type=jnp.float32)
          + bl_ref[...])
    ls = jnp.minimum(ls, MAX_LOGSTD)
    els = jnp.exp(ls)
    z = mu + eps_ref[...] * els
    z_ref[...] = z
    tot = jnp.sum(1.0 + 2.0 * ls - mu * mu - els * els)
    kl_ref[...] = jnp.reshape((-0.5 / (N * N)) * tot, (1, 1))


def _heads(h1, q0, q1, dinv, Wm0, Wm1, bm, Wl0, Wl1, bl, eps):
    HO = 2 * ZD
    full = lambda s: pl.BlockSpec(s, lambda: tuple(0 for _ in s))
    return pl.pallas_call(
        _heads_body,
        in_specs=[
            full((N, HO)), full((N, HO)), full((N, HO)), full((N, 1)),
            full((HO, ZD)), full((HO, ZD)), full((1, ZD)),
            full((HO, ZD)), full((HO, ZD)), full((1, ZD)),
            full((N, ZD)),
        ],
        out_specs=[full((N, ZD)), full((1, 1))],
        out_shape=[
            jax.ShapeDtypeStruct((N, ZD), jnp.float32),
            jax.ShapeDtypeStruct((1, 1), jnp.float32),
        ],
    )(h1, q0, q1, dinv, Wm0, Wm1, bm, Wl0, Wl1, bl, eps)


def _loss_body(qp_ref, qn_ref, kl_ref, loss_ref):
    i = pl.program_id(0)
    sp = jnp.sum(qp_ref[...], axis=1)
    sn = jnp.sum(qn_ref[...], axis=1)
    lp = jnp.log(jnp.maximum(jax.nn.sigmoid(sp) + EPS, EPS))
    ln = jnp.log(jnp.maximum(1.0 - jax.nn.sigmoid(sn) + EPS, EPS))
    part = jnp.reshape(-(jnp.sum(lp) + jnp.sum(ln)) / E, (1, 1))

    @pl.when(i == 0)
    def _():
        loss_ref[...] = kl_ref[...]

    loss_ref[...] += part


def _loss(qp, qn, kl):
    BE = 8000
    return pl.pallas_call(
        _loss_body,
        grid=(E // BE,),
        in_specs=[
            pl.BlockSpec((BE, L), lambda i: (i, 0)),
            pl.BlockSpec((BE, L), lambda i: (i, 0)),
            pl.BlockSpec((1, 1), lambda i: (0, 0)),
        ],
        out_specs=pl.BlockSpec((1, 1), lambda i: (0, 0)),
        out_shape=jax.ShapeDtypeStruct((1, 1), jnp.float32),
        compiler_params=pltpu.CompilerParams(
            dimension_semantics=("arbitrary",)),
    )(qp, qn, kl)


@jax.jit
def kernel(x_adj, edge_index, neg_edge_index, Wt, bt, W10, W11, b1,
           Wm0, Wm1, bm, Wl0, Wl1, bl):
    row = edge_index[0]
    col = edge_index[1]

    deg_part, remap = _edge_prep(row, col)
    degT = deg_part[:, :N, 0].T

    h, g, dinv = _big(x_adj, Wt, bt.reshape(1, HID), degT)

    zpad = jnp.zeros((8, HID), jnp.float32)
    tx1_part = _scatter(jnp.concatenate([g, zpad], axis=0), remap, col)

    h1, g1 = _mid(h, tx1_part[0], tx1_part[1], dinv,
                  W10, W11, b1.reshape(1, 2 * ZD))

    tx2_part = _scatter(jnp.concatenate([g1, zpad], axis=0), remap, col)

    eps = jax.random.normal(jax.random.key(12345), (N, ZD), jnp.float32)
    z, kl = _heads(h1, tx2_part[0, :, :2 * ZD], tx2_part[1, :, :2 * ZD],
                   dinv, Wm0, Wm1, bm.reshape(1, ZD),
                   Wl0, Wl1, bl.reshape(1, ZD), eps)

    z128 = jnp.concatenate([z, jnp.zeros((N, HID - ZD), jnp.float32)], axis=1)
    qp, qn = _recon(z128, edge_index[0], edge_index[1],
                    neg_edge_index[0], neg_edge_index[1])
    loss = _loss(qp, qn, kl)
    return z, loss[0, 0]


# speedup vs baseline: 7.2980x; 1.0233x over previous
import functools
import jax
import jax.numpy as jnp
from jax import lax
from jax.experimental import pallas as pl
from jax.experimental.pallas import tpu as pltpu
from jax.experimental.pallas import tpu_sc as plsc

N = 10000
E = 320000
HID = 128
ZD = 32
EPS = 1e-15
MAX_LOGSTD = 10.0

NC, NS, L = 2, 16, 16
NW = NC * NS
EPW = E // NW
CH = 80
NCH = EPW // CH
PADROW = N
NPAD_A = 10112
RB = 632
RB_LAST = N - (NS - 1) * RB


@functools.cache
def _sc_mesh():
    return plsc.VectorSubcoreMesh(core_axis_name="c", subcore_axis_name="s",
                                  num_cores=NC, num_subcores=NS)


def _fill_zeros8(zb):
    for i in range(8):
        for c in range(8):
            zb[i, pl.ds(c * L, L)] = jnp.zeros((L,), jnp.float32)


def _zero_rows(zb, acc, start, nrows):
    def cp(i, _):
        off = pl.multiple_of(start + i * 8, 8)
        pltpu.sync_copy(zb, acc.at[pl.ds(off, 8)])
        return _
    lax.fori_loop(0, nrows // 8, cp, None)


def _edge_prep_body(row_hbm, col_hbm, deg_hbm, remap_hbm,
                    rowv0, colv0, rmv0, rowv1, colv1, rmv1, rowv2, colv2, rmv2,
                    ones, zb, dacc,
                    si0, si1, si2, ss0, ss1, ss2, sr0, sr1, sr2):
    cid = lax.axis_index("c")
    sid = lax.axis_index("s")
    wid = sid * NC + cid
    RBA = NPAD_A // NS
    rowv = (rowv0, rowv1, rowv2)
    colv = (colv0, colv1, colv2)
    rmv = (rmv0, rmv1, rmv2)
    si = (si0, si1, si2); ss = (ss0, ss1, ss2); sr = (sr0, sr1, sr2)

    _fill_zeros8(zb)
    _zero_rows(zb, dacc, sid * RBA, RBA)

    def of(i, _):
        for c in range(8):
            ones[i, pl.ds(c * L, L)] = jnp.full((L,), 1.0, jnp.float32)
        return _
    lax.fori_loop(0, CH, of, None)
    plsc.subcore_barrier()

    def start_idx(j, b):
        base = wid * EPW + j * CH
        pltpu.async_copy(row_hbm.at[pl.ds(base, CH)], rowv[b], si[b])
        pltpu.async_copy(col_hbm.at[pl.ds(base, CH)], colv[b], si[b])

    def wait_idx(b):
        pltpu.make_async_copy(row_hbm.at[pl.ds(0, CH)], rowv[b], si[b]).wait()
        pltpu.make_async_copy(col_hbm.at[pl.ds(0, CH)], colv[b], si[b]).wait()

    def wait_scatter(b):
        pltpu.make_async_copy(ones, dacc.at[rmv[b]], ss[b]).wait()

    def wait_rstore(b):
        pltpu.make_async_copy(remap_hbm.at[pl.ds(0, CH)], rmv[b], sr[b]).wait()

    def compute(b):
        @plsc.parallel_loop(0, CH // L, unroll=5)
        def _(k):
            r = rowv[b][pl.ds(k * L, L)]
            c = colv[b][pl.ds(k * L, L)]
            rmv[b][pl.ds(k * L, L)] = jnp.where(r == c, PADROW, r).astype(jnp.int32)

    def emit(j, b):
        base = wid * EPW + j * CH
        pltpu.async_copy(ones, dacc.at[rmv[b]], ss[b], add=True)
        pltpu.async_copy(rmv[b], remap_hbm.at[pl.ds(base, CH)], sr[b])

    def unit(j, b, bp, bm, guard):
        wait_idx(b)
        wait_scatter(bm)
        wait_rstore(bm)
        if guard:
            @pl.when(j + 2 < NCH)
            def _():
                start_idx(j + 2, bm)
        else:
            start_idx(j + 2, bm)
        compute(b)
        emit(j, b)

    start_idx(0, 0)
    start_idx(1, 1)
    wait_idx(0)
    start_idx(2, 2)
    compute(0)
    emit(0, 0)
    wait_idx(1)
    wait_scatter(0)
    wait_rstore(0)
    start_idx(3, 0)
    compute(1)
    emit(1, 1)

    def triple(g, _):
        j = 2 + 3 * g
        unit(j, 2, 0, 1, guard=(True))
        unit(j + 1, 0, 1, 2, guard=True)
        unit(j + 2, 1, 2, 0, guard=True)
        return _
    lax.fori_loop(0, (NCH - 2) // 3, triple, None)
    wait_scatter((NCH - 1) % 3)
    wait_rstore((NCH - 1) % 3)

    plsc.subcore_barrier()
    off = pl.multiple_of(sid * RBA, 8)
    pltpu.sync_copy(dacc.at[pl.ds(off, RBA)], deg_hbm.at[cid, pl.ds(off, RBA)])


def _edge_prep(row, col):
    buf = lambda: [pltpu.VMEM((CH,), jnp.int32), pltpu.VMEM((CH,), jnp.int32),
                   pltpu.VMEM((CH,), jnp.int32)]
    return pl.kernel(
        _edge_prep_body,
        out_type=(
            jax.ShapeDtypeStruct((NC, NPAD_A, HID), jnp.float32),
            jax.ShapeDtypeStruct((E,), jnp.int32),
        ),
        mesh=_sc_mesh(),
        scratch_types=buf() + buf() + buf() + [
            pltpu.VMEM((CH, HID), jnp.float32),
            pltpu.VMEM((8, HID), jnp.float32),
            pltpu.VMEM_SHARED((NPAD_A, HID), jnp.float32),
        ] + [pltpu.SemaphoreType.DMA] * 9,
    )(row, col)


def _scatter_body(table_hbm, remap_hbm, col_hbm, out_hbm,
                  iv0, cv0, rows0, iv1, cv1, rows1, iv2, cv2, rows2,
                  zb, acc, si0, si1, si2, sg0, sg1, sg2, ss0, ss1, ss2):
    cid = lax.axis_index("c")
    sid = lax.axis_index("s")
    wid = sid * NC + cid
    iv = (iv0, iv1, iv2); cv = (cv0, cv1, cv2); rows = (rows0, rows1, rows2)
    si = (si0, si1, si2); sg = (sg0, sg1, sg2); ss = (ss0, ss1, ss2)

    _fill_zeros8(zb)

    @pl.when(sid < NS - 1)
    def _():
        _zero_rows(zb, acc, sid * RB, RB)

    @pl.when(sid == NS - 1)
    def _():
        _zero_rows(zb, acc, (NS - 1) * RB, RB_LAST)

    plsc.subcore_barrier()

    def start_idx(j, b):
        base = wid * EPW + j * CH
        pltpu.async_copy(remap_hbm.at[pl.ds(base, CH)], iv[b], si[b])
        pltpu.async_copy(col_hbm.at[pl.ds(base, CH)], cv[b], si[b])

    def wait_idx(b):
        pltpu.make_async_copy(remap_hbm.at[pl.ds(0, CH)], iv[b], si[b]).wait()
        pltpu.make_async_copy(col_hbm.at[pl.ds(0, CH)], cv[b], si[b]).wait()

    def wait_gather(b):
        pltpu.make_async_copy(table_hbm.at[iv[b]], rows[b], sg[b]).wait()

    def wait_scatter(b):
        pltpu.make_async_copy(rows[b], acc.at[cv[b]], ss[b]).wait()

    def unit(j, b, bp, bm, guard):
        wait_gather(b)
        wait_scatter(bm)
        if guard:
            @pl.when(j + 2 < NCH)
            def _():
                start_idx(j + 2, bm)
            @pl.when(j + 1 < NCH)
            def _():
                wait_idx(bp)
                pltpu.async_copy(table_hbm.at[iv[bp]], rows[bp], sg[bp])
        else:
            start_idx(j + 2, bm)
            wait_idx(bp)
            pltpu.async_copy(table_hbm.at[iv[bp]], rows[bp], sg[bp])
        pltpu.async_copy(rows[b], acc.at[cv[b]], ss[b], add=True)

    start_idx(0, 0)
    wait_idx(0)
    pltpu.async_copy(table_hbm.at[iv[0]], rows[0], sg[0])
    start_idx(1, 1)
    wait_gather(0)
    start_idx(2, 2)
    wait_idx(1)
    pltpu.async_copy(table_hbm.at[iv[1]], rows[1], sg[1])
    pltpu.async_copy(rows[0], acc.at[cv[0]], ss[0], add=True)
    wait_gather(1)
    wait_scatter(0)
    start_idx(3, 0)
    wait_idx(2)
    pltpu.async_copy(table_hbm.at[iv[2]], rows[2], sg[2])
    pltpu.async_copy(rows[1], acc.at[cv[1]], ss[1], add=True)

    def triple(g, _):
        j = 2 + 3 * g
        unit(j, 2, 0, 1, guard=(True))
        unit(j + 1, 0, 1, 2, guard=True)
        unit(j + 2, 1, 2, 0, guard=True)
        return _
    lax.fori_loop(0, (NCH - 2) // 3, triple, None)
    wait_scatter((NCH - 1) % 3)
    plsc.subcore_barrier()

    @pl.when(sid < NS - 1)
    def _():
        off = pl.multiple_of(sid * RB, 8)
        pltpu.sync_copy(acc.at[pl.ds(off, RB)], out_hbm.at[cid, pl.ds(off, RB)])

    @pl.when(sid == NS - 1)
    def _():
        off = (NS - 1) * RB
        pltpu.sync_copy(acc.at[pl.ds(off, RB_LAST)],
                        out_hbm.at[cid, pl.ds(off, RB_LAST)])


def _scatter(table_pad, remap, col):
    buf = lambda: [pltpu.VMEM((CH,), jnp.int32), pltpu.VMEM((CH,), jnp.int32),
                   pltpu.VMEM((CH, HID), jnp.float32)]
    return pl.kernel(
        _scatter_body,
        out_type=jax.ShapeDtypeStruct((NC, N, HID), jnp.float32),
        mesh=_sc_mesh(),
        scratch_types=buf() + buf() + buf() + [
            pltpu.VMEM((8, HID), jnp.float32),
            pltpu.VMEM_SHARED((N, HID), jnp.float32),
        ] + [pltpu.SemaphoreType.DMA] * 9,
    )(table_pad, remap, col)


def _recon_body(z_hbm, e0_hbm, e1_hbm, n0_hbm, n1_hbm, qp_hbm, qn_hbm,
                i0a, i1a, zaa, zba, qva, i0b, i1b, zab, zbb, qvb,
                i0c, i1c, zac, zbc, qvc,
                sia, sib, sic, sga, sgb, sgc, sqa, sqb, sqc):
    cid = lax.axis_index("c")
    sid = lax.axis_index("s")
    wid = sid * NC + cid
    i0 = (i0a, i0b, i0c); i1 = (i1a, i1b, i1c)
    za = (zaa, zab, zac); zb = (zba, zbb, zbc); qv = (qva, qvb, qvc)
    si = (sia, sib, sic); sg = (sga, sgb, sgc); sq = (sqa, sqb, sqc)

    def run_side(idx0_hbm, idx1_hbm, q_hbm):
        def start_idx(j, b):
            base = wid * EPW + j * CH
            pltpu.async_copy(idx0_hbm.at[pl.ds(base, CH)], i0[b], si[b])
            pltpu.async_copy(idx1_hbm.at[pl.ds(base, CH)], i1[b], si[b])

        def wait_idx(b):
            pltpu.make_async_copy(idx0_hbm.at[pl.ds(0, CH)], i0[b], si[b]).wait()
            pltpu.make_async_copy(idx1_hbm.at[pl.ds(0, CH)], i1[b], si[b]).wait()

        def start_gathers(b):
            pltpu.async_copy(z_hbm.at[i0[b]], za[b], sg[b])
            pltpu.async_copy(z_hbm.at[i1[b]], zb[b], sg[b])

        def wait_gathers(b):
            pltpu.make_async_copy(z_hbm.at[i0[b]], za[b], sg[b]).wait()
            pltpu.make_async_copy(z_hbm.at[i1[b]], zb[b], sg[b]).wait()

        def wait_qstore(b):
            pltpu.make_async_copy(q_hbm.at[pl.ds(0, CH)], qv[b], sq[b]).wait()

        def compute(j, b):
            for e in range(CH):
                a0 = za[b][e, pl.ds(0, L)]
                a1 = za[b][e, pl.ds(L, L)]
                c0 = zb[b][e, pl.ds(0, L)]
                c1 = zb[b][e, pl.ds(L, L)]
                qv[b][e, pl.ds(0, L)] = a0 * c0 + a1 * c1
            base = wid * EPW + j * CH
            pltpu.async_copy(qv[b], q_hbm.at[pl.ds(base, CH)], sq[b])

        def unit(j, b, bp, bm, guard):
            wait_gathers(b)
            wait_qstore(bm)
            if guard:
                @pl.when(j + 2 < NCH)
                def _():
                    start_idx(j + 2, bm)
                @pl.when(j + 1 < NCH)
                def _():
                    wait_idx(bp)
                    start_gathers(bp)
            else:
                start_idx(j + 2, bm)
                wait_idx(bp)
                start_gathers(bp)
            compute(j, b)

        start_idx(0, 0)
        wait_idx(0)
        start_gathers(0)
        start_idx(1, 1)
        wait_gathers(0)
        start_idx(2, 2)
        wait_idx(1)
        start_gathers(1)
        compute(0, 0)
        wait_gathers(1)
        wait_qstore(0)
        start_idx(3, 0)
        wait_idx(2)
        start_gathers(2)
        compute(1, 1)

        def triple(g, _):
            j = 2 + 3 * g
            unit(j, 2, 0, 1, guard=(True))
            unit(j + 1, 0, 1, 2, guard=True)
            unit(j + 2, 1, 2, 0, guard=True)
            return _
        lax.fori_loop(0, (NCH - 2) // 3, triple, None)
        wait_qstore((NCH - 1) % 3)

    run_side(e0_hbm, e1_hbm, qp_hbm)
    run_side(n0_hbm, n1_hbm, qn_hbm)


def _recon(z128, e0, e1, n0, n1):
    buf = lambda: [pltpu.VMEM((CH,), jnp.int32), pltpu.VMEM((CH,), jnp.int32),
                   pltpu.VMEM((CH, HID), jnp.float32), pltpu.VMEM((CH, HID), jnp.float32),
                   pltpu.VMEM((CH, L), jnp.float32)]
    return pl.kernel(
        _recon_body,
        out_type=(
            jax.ShapeDtypeStruct((E, L), jnp.float32),
            jax.ShapeDtypeStruct((E, L), jnp.float32),
        ),
        mesh=_sc_mesh(),
        scratch_types=buf() + buf() + buf() + [pltpu.SemaphoreType.DMA] * 9,
    )(z128, e0, e1, n0, n1)


def _big_body(x_ref, wt_ref, bt_ref, degt_ref, h_ref, g_ref, dinv_ref):
    acc = jax.lax.dot_general(
        x_ref[...], wt_ref[...], (((1,), (0,)), ((), ())),
        preferred_element_type=jnp.float32)
    deg = jnp.sum(degt_ref[...], axis=1, keepdims=True)
    pos = deg > 0
    dinv = jnp.where(pos, lax.rsqrt(jnp.where(pos, deg, 1.0)), 0.0)
    h = jax.nn.relu(acc + bt_ref[...])
    h_ref[...] = h
    g_ref[...] = h * dinv
    dinv_ref[...] = dinv


def _big(x_adj, Wt, bt, degT):
    BM = 400
    return pl.pallas_call(
        _big_body,
        grid=(N // BM,),
        in_specs=[
            pl.BlockSpec((BM, N), lambda i: (i, 0)),
            pl.BlockSpec((N, HID), lambda i: (0, 0)),
            pl.BlockSpec((1, HID), lambda i: (0, 0)),
            pl.BlockSpec((BM, NC), lambda i: (i, 0)),
        ],
        out_specs=[
            pl.BlockSpec((BM, HID), lambda i: (i, 0)),
            pl.BlockSpec((BM, HID), lambda i: (i, 0)),
            pl.BlockSpec((BM, 1), lambda i: (i, 0)),
        ],
        out_shape=[
            jax.ShapeDtypeStruct((N, HID), jnp.float32),
            jax.ShapeDtypeStruct((N, HID), jnp.float32),
            jax.ShapeDtypeStruct((N, 1), jnp.float32),
        ],
        compiler_params=pltpu.CompilerParams(
            dimension_semantics=("arbitrary",)),
    )(x_adj, Wt, bt, degT)


def _mid_body(h_ref, p0_ref, p1_ref, dinv_ref, w10_ref, w11_ref, b1_ref,
              h1_ref, g1_ref):
    HO = 2 * ZD
    dinv = dinv_ref[...]
    tx1 = -dinv * (p0_ref[...] + p1_ref[...])
    h1 = jnp.dot(h_ref[...], w10_ref[...], preferred_element_type=jnp.float32)
    h1 = h1 + jnp.dot(tx1, w11_ref[...], preferred_element_type=jnp.float32)
    h1 = jax.nn.relu(h1 + b1_ref[...])
    h1_ref[...] = h1
    g1_ref[:, :HO] = h1 * dinv
    g1_ref[:, HO:] = jnp.zeros_like(g1_ref[:, HO:])


def _mid(h, p0, p1, dinv, W10, W11, b1):
    BM = 1000
    HO = 2 * ZD
    return pl.pallas_call(
        _mid_body,
        grid=(N // BM,),
        in_specs=[
            pl.BlockSpec((BM, HID), lambda i: (i, 0)),
            pl.BlockSpec((BM, HID), lambda i: (i, 0)),
            pl.BlockSpec((BM, HID), lambda i: (i, 0)),
            pl.BlockSpec((BM, 1), lambda i: (i, 0)),
            pl.BlockSpec((HID, HO), lambda i: (0, 0)),
            pl.BlockSpec((HID, HO), lambda i: (0, 0)),
            pl.BlockSpec((1, HO), lambda i: (0, 0)),
        ],
        out_specs=[
            pl.BlockSpec((BM, HO), lambda i: (i, 0)),
            pl.BlockSpec((BM, HID), lambda i: (i, 0)),
        ],
        out_shape=[
            jax.ShapeDtypeStruct((N, HO), jnp.float32),
            jax.ShapeDtypeStruct((N, HID), jnp.float32),
        ],
        compiler_params=pltpu.CompilerParams(
            dimension_semantics=("arbitrary",)),
    )(h, p0, p1, dinv, W10, W11, b1)


def _heads_body(h1_ref, q0_ref, q1_ref, dinv_ref, wm0_ref, wm1_ref, bm_ref,
                wl0_ref, wl1_ref, bl_ref, eps_ref, z_ref, zb_ref, kl_ref):
    tx2 = -dinv_ref[...] * (q0_ref[...] + q1_ref[...])
    h1 = h1_ref[...]
    mu = (jnp.dot(h1, wm0_ref[...], preferred_element_type=jnp.float32)
          + jnp.dot(tx2, wm1_ref[...], preferred_element_type=jnp.float32)
          + bm_ref[...])
    ls = (jnp.dot(h1, wl0_ref[...], preferred_element_type=jnp.float32)
          + jnp.dot(tx2, wl1_ref[...], preferred_element_type=jnp.float32)
          + bl_ref[...])
    ls = jnp.minimum(ls, MAX_LOGSTD)
    els = jnp.exp(ls)
    z = mu + eps_ref[...] * els
    z_ref[...] = z
    zb_ref[:, :ZD] = z
    zb_ref[:, ZD:] = jnp.zeros_like(zb_ref[:, ZD:])
    tot = jnp.sum(1.0 + 2.0 * ls - mu * mu - els * els)
    kl_ref[...] = jnp.reshape((-0.5 / (N * N)) * tot, (1, 1))


def _heads(h1, q0, q1, dinv, Wm0, Wm1, bm, Wl0, Wl1, bl, eps):
    HO = 2 * ZD
    full = lambda s: pl.BlockSpec(s, lambda: tuple(0 for _ in s))
    return pl.pallas_call(
        _heads_body,
        in_specs=[
            full((N, HO)), full((N, HO)), full((N, HO)), full((N, 1)),
            full((HO, ZD)), full((HO, ZD)), full((1, ZD)),
            full((HO, ZD)), full((HO, ZD)), full((1, ZD)),
            full((N, ZD)),
        ],
        out_specs=[full((N, ZD)), full((N, HID)), full((1, 1))],
        out_shape=[
            jax.ShapeDtypeStruct((N, ZD), jnp.float32),
            jax.ShapeDtypeStruct((N, HID), jnp.float32),
            jax.ShapeDtypeStruct((1, 1), jnp.float32),
        ],
    )(h1, q0, q1, dinv, Wm0, Wm1, bm, Wl0, Wl1, bl, eps)


def _loss_body(qp_ref, qn_ref, kl_ref, loss_ref):
    i = pl.program_id(0)
    sp = jnp.sum(qp_ref[...], axis=1)
    sn = jnp.sum(qn_ref[...], axis=1)
    lp = jnp.log(jnp.maximum(jax.nn.sigmoid(sp) + EPS, EPS))
    ln = jnp.log(jnp.maximum(1.0 - jax.nn.sigmoid(sn) + EPS, EPS))
    part = jnp.reshape(-(jnp.sum(lp) + jnp.sum(ln)) / E, (1, 1))

    @pl.when(i == 0)
    def _():
        loss_ref[...] = kl_ref[...]

    loss_ref[...] += part


def _loss(qp, qn, kl):
    BE = 8000
    return pl.pallas_call(
        _loss_body,
        grid=(E // BE,),
        in_specs=[
            pl.BlockSpec((BE, L), lambda i: (i, 0)),
            pl.BlockSpec((BE, L), lambda i: (i, 0)),
            pl.BlockSpec((1, 1), lambda i: (0, 0)),
        ],
        out_specs=pl.BlockSpec((1, 1), lambda i: (0, 0)),
        out_shape=jax.ShapeDtypeStruct((1, 1), jnp.float32),
        compiler_params=pltpu.CompilerParams(
            dimension_semantics=("arbitrary",)),
    )(qp, qn, kl)


@jax.jit
def kernel(x_adj, edge_index, neg_edge_index, Wt, bt, W10, W11, b1,
           Wm0, Wm1, bm, Wl0, Wl1, bl):
    row = edge_index[0]
    col = edge_index[1]

    deg_part, remap = _edge_prep(row, col)
    degT = deg_part[:, :N, 0].T

    h, g, dinv = _big(x_adj, Wt, bt.reshape(1, HID), degT)

    zpad = jnp.zeros((8, HID), jnp.float32)
    tx1_part = _scatter(jnp.concatenate([g, zpad], axis=0), remap, col)

    h1, g1 = _mid(h, tx1_part[0], tx1_part[1], dinv,
                  W10, W11, b1.reshape(1, 2 * ZD))

    tx2_part = _scatter(jnp.concatenate([g1, zpad], axis=0), remap, col)

    eps = jax.random.normal(jax.random.key(12345), (N, ZD), jnp.float32)
    z, zb128, kl = _heads(h1, tx2_part[0, :, :2 * ZD], tx2_part[1, :, :2 * ZD],
                          dinv, Wm0, Wm1, bm.reshape(1, ZD),
                          Wl0, Wl1, bl.reshape(1, ZD), eps)

    qp, qn = _recon(zb128, edge_index[0], edge_index[1],
                    neg_edge_index[0], neg_edge_index[1])
    loss = _loss(qp, qn, kl)
    return z, loss[0, 0]


# speedup vs baseline: 7.2997x; 1.0002x over previous
import functools
import jax
import jax.numpy as jnp
from jax import lax
from jax.experimental import pallas as pl
from jax.experimental.pallas import tpu as pltpu
from jax.experimental.pallas import tpu_sc as plsc

N = 10000
E = 320000
HID = 128
ZD = 32
EPS = 1e-15
MAX_LOGSTD = 10.0

NC, NS, L = 2, 16, 16
NW = NC * NS
EPW = E // NW
CH = 80
NCH = EPW // CH
PADROW = N
NPAD_A = 10112
RB = 632
RB_LAST = N - (NS - 1) * RB


@functools.cache
def _sc_mesh():
    return plsc.VectorSubcoreMesh(core_axis_name="c", subcore_axis_name="s",
                                  num_cores=NC, num_subcores=NS)


def _fill_zeros8(zb):
    for i in range(8):
        for c in range(8):
            zb[i, pl.ds(c * L, L)] = jnp.zeros((L,), jnp.float32)


def _zero_rows(zb, acc, start, nrows):
    def cp(i, _):
        off = pl.multiple_of(start + i * 8, 8)
        pltpu.sync_copy(zb, acc.at[pl.ds(off, 8)])
        return _
    lax.fori_loop(0, nrows // 8, cp, None)


def _edge_prep_body(row_hbm, col_hbm, deg_hbm, remap_hbm,
                    rowv0, colv0, rmv0, rowv1, colv1, rmv1, rowv2, colv2, rmv2,
                    ones, zb, dacc,
                    si0, si1, si2, ss0, ss1, ss2, sr0, sr1, sr2):
    cid = lax.axis_index("c")
    sid = lax.axis_index("s")
    wid = sid * NC + cid
    RBA = NPAD_A // NS
    rowv = (rowv0, rowv1, rowv2)
    colv = (colv0, colv1, colv2)
    rmv = (rmv0, rmv1, rmv2)
    si = (si0, si1, si2); ss = (ss0, ss1, ss2); sr = (sr0, sr1, sr2)

    _fill_zeros8(zb)
    _zero_rows(zb, dacc, sid * RBA, RBA)

    def of(i, _):
        for c in range(8):
            ones[i, pl.ds(c * L, L)] = jnp.full((L,), 1.0, jnp.float32)
        return _
    lax.fori_loop(0, CH, of, None)
    plsc.subcore_barrier()

    def start_idx(j, b):
        base = wid * EPW + j * CH
        pltpu.async_copy(row_hbm.at[pl.ds(base, CH)], rowv[b], si[b])
        pltpu.async_copy(col_hbm.at[pl.ds(base, CH)], colv[b], si[b])

    def wait_idx(b):
        pltpu.make_async_copy(row_hbm.at[pl.ds(0, CH)], rowv[b], si[b]).wait()
        pltpu.make_async_copy(col_hbm.at[pl.ds(0, CH)], colv[b], si[b]).wait()

    def wait_scatter(b):
        pltpu.make_async_copy(ones, dacc.at[rmv[b]], ss[b]).wait()

    def wait_rstore(b):
        pltpu.make_async_copy(remap_hbm.at[pl.ds(0, CH)], rmv[b], sr[b]).wait()

    def compute(b):
        @plsc.parallel_loop(0, CH // L, unroll=5)
        def _(k):
            r = rowv[b][pl.ds(k * L, L)]
            c = colv[b][pl.ds(k * L, L)]
            rmv[b][pl.ds(k * L, L)] = jnp.where(r == c, PADROW, r).astype(jnp.int32)

    def emit(j, b):
        base = wid * EPW + j * CH
        pltpu.async_copy(ones, dacc.at[rmv[b]], ss[b], add=True)
        pltpu.async_copy(rmv[b], remap_hbm.at[pl.ds(base, CH)], sr[b])

    def unit(j, b, bp, bm, guard):
        wait_idx(b)
        wait_scatter(bm)
        wait_rstore(bm)
        if guard:
            @pl.when(j + 2 < NCH)
            def _():
                start_idx(j + 2, bm)
        else:
            start_idx(j + 2, bm)
        compute(b)
        emit(j, b)

    start_idx(0, 0)
    start_idx(1, 1)
    wait_idx(0)
    start_idx(2, 2)
    compute(0)
    emit(0, 0)
    wait_idx(1)
    wait_scatter(0)
    wait_rstore(0)
    start_idx(3, 0)
    compute(1)
    emit(1, 1)

    def triple(g, _):
        j = 2 + 3 * g
        unit(j, 2, 0, 1, guard=(True))
        unit(j + 1, 0, 1, 2, guard=True)
        unit(j + 2, 1, 2, 0, guard=True)
        return _
    lax.fori_loop(0, (NCH - 2) // 3, triple, None)
    wait_scatter((NCH - 1) % 3)
    wait_rstore((NCH - 1) % 3)

    plsc.subcore_barrier()
    off = pl.multiple_of(sid * RBA, 8)
    pltpu.sync_copy(dacc.at[pl.ds(off, RBA)], deg_hbm.at[cid, pl.ds(off, RBA)])


def _edge_prep(row, col):
    buf = lambda: [pltpu.VMEM((CH,), jnp.int32), pltpu.VMEM((CH,), jnp.int32),
                   pltpu.VMEM((CH,), jnp.int32)]
    return pl.kernel(
        _edge_prep_body,
        out_type=(
            jax.ShapeDtypeStruct((NC, NPAD_A, HID), jnp.float32),
            jax.ShapeDtypeStruct((E,), jnp.int32),
        ),
        mesh=_sc_mesh(),
        scratch_types=buf() + buf() + buf() + [
            pltpu.VMEM((CH, HID), jnp.float32),
            pltpu.VMEM((8, HID), jnp.float32),
            pltpu.VMEM_SHARED((NPAD_A, HID), jnp.float32),
        ] + [pltpu.SemaphoreType.DMA] * 9,
    )(row, col)


def _scatter_body(table_hbm, remap_hbm, col_hbm, out_hbm,
                  iv0, cv0, rows0, iv1, cv1, rows1, iv2, cv2, rows2,
                  zb, acc, si0, si1, si2, sg0, sg1, sg2, ss0, ss1, ss2):
    cid = lax.axis_index("c")
    sid = lax.axis_index("s")
    wid = sid * NC + cid
    iv = (iv0, iv1, iv2); cv = (cv0, cv1, cv2); rows = (rows0, rows1, rows2)
    si = (si0, si1, si2); sg = (sg0, sg1, sg2); ss = (ss0, ss1, ss2)

    _fill_zeros8(zb)

    @pl.when(sid < NS - 1)
    def _():
        _zero_rows(zb, acc, sid * RB, RB)

    @pl.when(sid == NS - 1)
    def _():
        _zero_rows(zb, acc, (NS - 1) * RB, RB_LAST)

    plsc.subcore_barrier()

    def start_idx(j, b):
        base = wid * EPW + j * CH
        pltpu.async_copy(remap_hbm.at[pl.ds(base, CH)], iv[b], si[b])
        pltpu.async_copy(col_hbm.at[pl.ds(base, CH)], cv[b], si[b])

    def wait_idx(b):
        pltpu.make_async_copy(remap_hbm.at[pl.ds(0, CH)], iv[b], si[b]).wait()
        pltpu.make_async_copy(col_hbm.at[pl.ds(0, CH)], cv[b], si[b]).wait()

    def wait_gather(b):
        pltpu.make_async_copy(table_hbm.at[iv[b]], rows[b], sg[b]).wait()

    def wait_scatter(b):
        pltpu.make_async_copy(rows[b], acc.at[cv[b]], ss[b]).wait()

    def unit(j, b, bp, bm, guard):
        wait_gather(b)
        wait_scatter(bm)
        if guard:
            @pl.when(j + 2 < NCH)
            def _():
                start_idx(j + 2, bm)
            @pl.when(j + 1 < NCH)
            def _():
                wait_idx(bp)
                pltpu.async_copy(table_hbm.at[iv[bp]], rows[bp], sg[bp])
        else:
            start_idx(j + 2, bm)
            wait_idx(bp)
            pltpu.async_copy(table_hbm.at[iv[bp]], rows[bp], sg[bp])
        pltpu.async_copy(rows[b], acc.at[cv[b]], ss[b], add=True)

    start_idx(0, 0)
    wait_idx(0)
    pltpu.async_copy(table_hbm.at[iv[0]], rows[0], sg[0])
    start_idx(1, 1)
    wait_gather(0)
    start_idx(2, 2)
    wait_idx(1)
    pltpu.async_copy(table_hbm.at[iv[1]], rows[1], sg[1])
    pltpu.async_copy(rows[0], acc.at[cv[0]], ss[0], add=True)
    wait_gather(1)
    wait_scatter(0)
    start_idx(3, 0)
    wait_idx(2)
    pltpu.async_copy(table_hbm.at[iv[2]], rows[2], sg[2])
    pltpu.async_copy(rows[1], acc.at[cv[1]], ss[1], add=True)

    def triple(g, _):
        j = 2 + 3 * g
        unit(j, 2, 0, 1, guard=(True))
        unit(j + 1, 0, 1, 2, guard=True)
        unit(j + 2, 1, 2, 0, guard=True)
        return _
    lax.fori_loop(0, (NCH - 2) // 3, triple, None)
    wait_scatter((NCH - 1) % 3)
    plsc.subcore_barrier()

    @pl.when(sid < NS - 1)
    def _():
        off = pl.multiple_of(sid * RB, 8)
        pltpu.sync_copy(acc.at[pl.ds(off, RB)], out_hbm.at[cid, pl.ds(off, RB)])

    @pl.when(sid == NS - 1)
    def _():
        off = (NS - 1) * RB
        pltpu.sync_copy(acc.at[pl.ds(off, RB_LAST)],
                        out_hbm.at[cid, pl.ds(off, RB_LAST)])


def _scatter(table_pad, remap, col):
    buf = lambda: [pltpu.VMEM((CH,), jnp.int32), pltpu.VMEM((CH,), jnp.int32),
                   pltpu.VMEM((CH, HID), jnp.float32)]
    return pl.kernel(
        _scatter_body,
        out_type=jax.ShapeDtypeStruct((NC, N, HID), jnp.float32),
        mesh=_sc_mesh(),
        scratch_types=buf() + buf() + buf() + [
            pltpu.VMEM((8, HID), jnp.float32),
            pltpu.VMEM_SHARED((N, HID), jnp.float32),
        ] + [pltpu.SemaphoreType.DMA] * 9,
    )(table_pad, remap, col)


def _recon_body(z_hbm, e0_hbm, e1_hbm, n0_hbm, n1_hbm, qp_hbm, qn_hbm,
                i0a, i1a, zaa, zba, qva, i0b, i1b, zab, zbb, qvb,
                i0c, i1c, zac, zbc, qvc,
                sia, sib, sic, sga, sgb, sgc, sqa, sqb, sqc):
    cid = lax.axis_index("c")
    sid = lax.axis_index("s")
    wid = sid * NC + cid
    i0 = (i0a, i0b, i0c); i1 = (i1a, i1b, i1c)
    za = (zaa, zab, zac); zb = (zba, zbb, zbc); qv = (qva, qvb, qvc)
    si = (sia, sib, sic); sg = (sga, sgb, sgc); sq = (sqa, sqb, sqc)

    def run_side(idx0_hbm, idx1_hbm, q_hbm):
        def start_idx(j, b):
            base = wid * EPW + j * CH
            pltpu.async_copy(idx0_hbm.at[pl.ds(base, CH)], i0[b], si[b])
            pltpu.async_copy(idx1_hbm.at[pl.ds(base, CH)], i1[b], si[b])

        def wait_idx(b):
            pltpu.make_async_copy(idx0_hbm.at[pl.ds(0, CH)], i0[b], si[b]).wait()
            pltpu.make_async_copy(idx1_hbm.at[pl.ds(0, CH)], i1[b], si[b]).wait()

        def start_gathers(b):
            pltpu.async_copy(z_hbm.at[i0[b]], za[b], sg[b])
            pltpu.async_copy(z_hbm.at[i1[b]], zb[b], sg[b])

        def wait_gathers(b):
            pltpu.make_async_copy(z_hbm.at[i0[b]], za[b], sg[b]).wait()
            pltpu.make_async_copy(z_hbm.at[i1[b]], zb[b], sg[b]).wait()

        def wait_qstore(b):
            pltpu.make_async_copy(q_hbm.at[pl.ds(0, CH)], qv[b], sq[b]).wait()

        def compute(j, b):
            for e in range(CH):
                a0 = za[b][e, pl.ds(0, L)]
                a1 = za[b][e, pl.ds(L, L)]
                c0 = zb[b][e, pl.ds(0, L)]
                c1 = zb[b][e, pl.ds(L, L)]
                qv[b][e, pl.ds(0, L)] = a0 * c0 + a1 * c1
            base = wid * EPW + j * CH
            pltpu.async_copy(qv[b], q_hbm.at[pl.ds(base, CH)], sq[b])

        def unit(j, b, bp, bm, guard):
            wait_gathers(b)
            wait_qstore(bm)
            if guard:
                @pl.when(j + 2 < NCH)
                def _():
                    start_idx(j + 2, bm)
                @pl.when(j + 1 < NCH)
                def _():
                    wait_idx(bp)
                    start_gathers(bp)
            else:
                start_idx(j + 2, bm)
                wait_idx(bp)
                start_gathers(bp)
            compute(j, b)

        start_idx(0, 0)
        wait_idx(0)
        start_gathers(0)
        start_idx(1, 1)
        wait_gathers(0)
        start_idx(2, 2)
        wait_idx(1)
        start_gathers(1)
        compute(0, 0)
        wait_gathers(1)
        wait_qstore(0)
        start_idx(3, 0)
        wait_idx(2)
        start_gathers(2)
        compute(1, 1)

        def triple(g, _):
            j = 2 + 3 * g
            unit(j, 2, 0, 1, guard=(True))
            unit(j + 1, 0, 1, 2, guard=True)
            unit(j + 2, 1, 2, 0, guard=True)
            return _
        lax.fori_loop(0, (NCH - 2) // 3, triple, None)
        wait_qstore((NCH - 1) % 3)

    run_side(e0_hbm, e1_hbm, qp_hbm)
    run_side(n0_hbm, n1_hbm, qn_hbm)


def _recon(z128, e0, e1, n0, n1):
    buf = lambda: [pltpu.VMEM((CH,), jnp.int32), pltpu.VMEM((CH,), jnp.int32),
                   pltpu.VMEM((CH, HID), jnp.float32), pltpu.VMEM((CH, HID), jnp.float32),
                   pltpu.VMEM((CH, L), jnp.float32)]
    return pl.kernel(
        _recon_body,
        out_type=(
            jax.ShapeDtypeStruct((E, L), jnp.float32),
            jax.ShapeDtypeStruct((E, L), jnp.float32),
        ),
        mesh=_sc_mesh(),
        scratch_types=buf() + buf() + buf() + [pltpu.SemaphoreType.DMA] * 9,
    )(z128, e0, e1, n0, n1)


def _big_body(x_ref, wt_ref, bt_ref, degt_ref, h_ref, g_ref, dinv_ref):
    acc = jax.lax.dot_general(
        x_ref[...], wt_ref[...], (((1,), (0,)), ((), ())),
        preferred_element_type=jnp.float32)
    deg = jnp.sum(degt_ref[...], axis=1, keepdims=True)
    pos = deg > 0
    dinv = jnp.where(pos, lax.rsqrt(jnp.where(pos, deg, 1.0)), 0.0)
    h = jax.nn.relu(acc + bt_ref[...])
    h_ref[...] = h
    g_ref[...] = h * dinv
    dinv_ref[...] = dinv


def _big(x_adj, Wt, bt, degT):
    BM = 200
    return pl.pallas_call(
        _big_body,
        grid=(N // BM,),
        in_specs=[
            pl.BlockSpec((BM, N), lambda i: (i, 0)),
            pl.BlockSpec((N, HID), lambda i: (0, 0)),
            pl.BlockSpec((1, HID), lambda i: (0, 0)),
            pl.BlockSpec((BM, NC), lambda i: (i, 0)),
        ],
        out_specs=[
            pl.BlockSpec((BM, HID), lambda i: (i, 0)),
            pl.BlockSpec((BM, HID), lambda i: (i, 0)),
            pl.BlockSpec((BM, 1), lambda i: (i, 0)),
        ],
        out_shape=[
            jax.ShapeDtypeStruct((N, HID), jnp.float32),
            jax.ShapeDtypeStruct((N, HID), jnp.float32),
            jax.ShapeDtypeStruct((N, 1), jnp.float32),
        ],
        compiler_params=pltpu.CompilerParams(
            dimension_semantics=("arbitrary",)),
    )(x_adj, Wt, bt, degT)


def _mid_body(h_ref, p0_ref, p1_ref, dinv_ref, w10_ref, w11_ref, b1_ref,
              h1_ref, g1_ref):
    HO = 2 * ZD
    dinv = dinv_ref[...]
    tx1 = -dinv * (p0_ref[...] + p1_ref[...])
    h1 = jnp.dot(h_ref[...], w10_ref[...], preferred_element_type=jnp.float32)
    h1 = h1 + jnp.dot(tx1, w11_ref[...], preferred_element_type=jnp.float32)
    h1 = jax.nn.relu(h1 + b1_ref[...])
    h1_ref[...] = h1
    g1_ref[:, :HO] = h1 * dinv
    g1_ref[:, HO:] = jnp.zeros_like(g1_ref[:, HO:])


def _mid(h, p0, p1, dinv, W10, W11, b1):
    BM = 1000
    HO = 2 * ZD
    return pl.pallas_call(
        _mid_body,
        grid=(N // BM,),
        in_specs=[
            pl.BlockSpec((BM, HID), lambda i: (i, 0)),
            pl.BlockSpec((BM, HID), lambda i: (i, 0)),
            pl.BlockSpec((BM, HID), lambda i: (i, 0)),
            pl.BlockSpec((BM, 1), lambda i: (i, 0)),
            pl.BlockSpec((HID, HO), lambda i: (0, 0)),
            pl.BlockSpec((HID, HO), lambda i: (0, 0)),
            pl.BlockSpec((1, HO), lambda i: (0, 0)),
        ],
        out_specs=[
            pl.BlockSpec((BM, HO), lambda i: (i, 0)),
            pl.BlockSpec((BM, HID), lambda i: (i, 0)),
        ],
        out_shape=[
            jax.ShapeDtypeStruct((N, HO), jnp.float32),
            jax.ShapeDtypeStruct((N, HID), jnp.float32),
        ],
        compiler_params=pltpu.CompilerParams(
            dimension_semantics=("arbitrary",)),
    )(h, p0, p1, dinv, W10, W11, b1)


def _heads_body(h1_ref, q0_ref, q1_ref, dinv_ref, wm0_ref, wm1_ref, bm_ref,
                wl0_ref, wl1_ref, bl_ref, eps_ref, z_ref, zb_ref, kl_ref):
    tx2 = -dinv_ref[...] * (q0_ref[...] + q1_ref[...])
    h1 = h1_ref[...]
    mu = (jnp.dot(h1, wm0_ref[...], preferred_element_type=jnp.float32)
          + jnp.dot(tx2, wm1_ref[...], preferred_element_type=jnp.float32)
          + bm_ref[...])
    ls = (jnp.dot(h1, wl0_ref[...], preferred_element_type=jnp.float32)
          + jnp.dot(tx2, wl1_ref[...], preferred_element_type=jnp.float32)
          + bl_ref[...])
    ls = jnp.minimum(ls, MAX_LOGSTD)
    els = jnp.exp(ls)
    z = mu + eps_ref[...] * els
    z_ref[...] = z
    zb_ref[:, :ZD] = z
    zb_ref[:, ZD:] = jnp.zeros_like(zb_ref[:, ZD:])
    tot = jnp.sum(1.0 + 2.0 * ls - mu * mu - els * els)
    kl_ref[...] = jnp.reshape((-0.5 / (N * N)) * tot, (1, 1))


def _heads(h1, q0, q1, dinv, Wm0, Wm1, bm, Wl0, Wl1, bl, eps):
    HO = 2 * ZD
    full = lambda s: pl.BlockSpec(s, lambda: tuple(0 for _ in s))
    return pl.pallas_call(
        _heads_body,
        in_specs=[
            full((N, HO)), full((N, HO)), full((N, HO)), full((N, 1)),
            full((HO, ZD)), full((HO, ZD)), full((1, ZD)),
            full((HO, ZD)), full((HO, ZD)), full((1, ZD)),
            full((N, ZD)),
        ],
        out_specs=[full((N, ZD)), full((N, HID)), full((1, 1))],
        out_shape=[
            jax.ShapeDtypeStruct((N, ZD), jnp.float32),
            jax.ShapeDtypeStruct((N, HID), jnp.float32),
            jax.ShapeDtypeStruct((1, 1), jnp.float32),
        ],
    )(h1, q0, q1, dinv, Wm0, Wm1, bm, Wl0, Wl1, bl, eps)


def _loss_body(qp_ref, qn_ref, kl_ref, loss_ref):
    i = pl.program_id(0)
    sp = jnp.sum(qp_ref[...], axis=1)
    sn = jnp.sum(qn_ref[...], axis=1)
    lp = jnp.log(jnp.maximum(jax.nn.sigmoid(sp) + EPS, EPS))
    ln = jnp.log(jnp.maximum(1.0 - jax.nn.sigmoid(sn) + EPS, EPS))
    part = jnp.reshape(-(jnp.sum(lp) + jnp.sum(ln)) / E, (1, 1))

    @pl.when(i == 0)
    def _():
        loss_ref[...] = kl_ref[...]

    loss_ref[...] += part


def _loss(qp, qn, kl):
    BE = 8000
    return pl.pallas_call(
        _loss_body,
        grid=(E // BE,),
        in_specs=[
            pl.BlockSpec((BE, L), lambda i: (i, 0)),
            pl.BlockSpec((BE, L), lambda i: (i, 0)),
            pl.BlockSpec((1, 1), lambda i: (0, 0)),
        ],
        out_specs=pl.BlockSpec((1, 1), lambda i: (0, 0)),
        out_shape=jax.ShapeDtypeStruct((1, 1), jnp.float32),
        compiler_params=pltpu.CompilerParams(
            dimension_semantics=("arbitrary",)),
    )(qp, qn, kl)


@jax.jit
def kernel(x_adj, edge_index, neg_edge_index, Wt, bt, W10, W11, b1,
           Wm0, Wm1, bm, Wl0, Wl1, bl):
    row = edge_index[0]
    col = edge_index[1]

    deg_part, remap = _edge_prep(row, col)
    degT = deg_part[:, :N, 0].T

    h, g, dinv = _big(x_adj, Wt, bt.reshape(1, HID), degT)

    zpad = jnp.zeros((8, HID), jnp.float32)
    tx1_part = _scatter(jnp.concatenate([g, zpad], axis=0), remap, col)

    h1, g1 = _mid(h, tx1_part[0], tx1_part[1], dinv,
                  W10, W11, b1.reshape(1, 2 * ZD))

    tx2_part = _scatter(jnp.concatenate([g1, zpad], axis=0), remap, col)

    eps = jax.random.normal(jax.random.key(12345), (N, ZD), jnp.float32)
    z, zb128, kl = _heads(h1, tx2_part[0, :, :2 * ZD], tx2_part[1, :, :2 * ZD],
                          dinv, Wm0, Wm1, bm.reshape(1, ZD),
                          Wl0, Wl1, bl.reshape(1, ZD), eps)

    qp, qn = _recon(zb128, edge_index[0], edge_index[1],
                    neg_edge_index[0], neg_edge_index[1])
    loss = _loss(qp, qn, kl)
    return z, loss[0, 0]
